# Initial kernel scaffold; baseline (speedup 1.0000x reference)
import jax
import jax.numpy as jnp
from jax.experimental import pallas as pl


def kernel(x, str_init, edge_index, tep_out, W_emb, b_emb, W_gcn0, b_gcn0, W_gcn1, b_gcn1, W_out, b_out, W_gat, att_src, att_dst, b_gat):
    raise NotImplementedError("write your pallas kernel here")


# speedup vs baseline: 8.1021x; 8.1021x over previous
import functools

import jax
import jax.numpy as jnp
from jax.experimental import pallas as pl
from jax.experimental.pallas import tpu as pltpu

N_NODES = 10000
N_EDGES = 320000
BATCH = 4
HID = 128
PRED = 12
HEADS = 3
HF = HEADS * PRED

_ROWS = 1000


def _prep_gcn_body(x_ref, dinv_ref, We_ref, be_ref, W0_ref, h0_ref, g0_ref):
    s0 = jnp.dot(x_ref[...], We_ref[...], preferred_element_type=jnp.float32)
    s0 = s0 + be_ref[...]
    h0 = jnp.dot(s0, W0_ref[...], preferred_element_type=jnp.float32)
    h0_ref[...] = h0
    g0_ref[...] = h0 * dinv_ref[...]


def _gcn_step_body(acc_ref, h_ref, dinv_ref, b_ref, Wn_ref, hn_ref, gn_ref):
    dinv = dinv_ref[...]
    s = jnp.tanh(dinv * acc_ref[...] + dinv * dinv * h_ref[...] + b_ref[...])
    hn = jnp.dot(s, Wn_ref[...], preferred_element_type=jnp.float32)
    hn_ref[...] = hn
    gn_ref[...] = hn * dinv


def _gcn_final_body(acc_ref, h_ref, dinv_ref, b_ref, Wo_ref, bo_ref, out_ref):
    dinv = dinv_ref[...]
    s = jnp.tanh(dinv * acc_ref[...] + dinv * dinv * h_ref[...] + b_ref[...])
    out_ref[...] = jnp.dot(s, Wo_ref[...], preferred_element_type=jnp.float32) + bo_ref[...]


def _gat_prep_body(x_ref, Wg_ref, As_ref, Ad_ref, h_ref, exs_ref, asrc_ref, adst_ref):
    h = jnp.dot(x_ref[...], Wg_ref[...], preferred_element_type=jnp.float32)
    h_ref[...] = h
    a_s = jnp.dot(h, As_ref[...], preferred_element_type=jnp.float32)
    a_d = jnp.dot(h, Ad_ref[...], preferred_element_type=jnp.float32)
    asrc_ref[...] = a_s
    adst_ref[...] = a_d
    z = a_s + a_d
    exs_ref[...] = jnp.exp(jnp.where(z > 0, z, 0.2 * z))


def _gat_combine_body(h_ref, exs_ref, num_ref, den_ref, R_ref, M_ref, b_ref,
                      out_ref):
    i = pl.program_id(0)
    real = (i < N_NODES // _ROWS).astype(jnp.float32)
    h = h_ref[...]
    exs3 = exs_ref[...]
    ex36 = jnp.dot(exs3, R_ref[...], preferred_element_type=jnp.float32)
    num = num_ref[...] * real + h * ex36
    den3 = den_ref[...] * real + exs3
    den36 = jnp.dot(den3, R_ref[...], preferred_element_type=jnp.float32) + 1e-16
    ratio = num / den36
    out_ref[...] = (
        jnp.dot(ratio, M_ref[...], preferred_element_type=jnp.float32) / HEADS
        + b_ref[...]
    )


def _full(shape):
    return pl.BlockSpec(shape, lambda i: (0,) * len(shape))


def _rows(width, blk=_ROWS):
    return pl.BlockSpec((blk, width), lambda i: (i, 0))


def _rows_clamped(width, nblk, blk=_ROWS):
    return pl.BlockSpec((blk, width), lambda i: (jnp.minimum(i, nblk - 1), 0))


@jax.jit
def _run(x, str_init, edge_index, tep_out, W_emb, b_emb, W_gcn0, b_gcn0,
         W_gcn1, b_gcn1, W_out, b_out, W_gat, att_src, att_dst, b_gat):
    src = edge_index[0].astype(jnp.int32)
    dst = edge_index[1].astype(jnp.int32)

    deg = jax.ops.segment_sum(jnp.ones(N_EDGES, jnp.float32), dst,
                              num_segments=N_NODES) + 1.0
    dinv = jax.lax.rsqrt(deg)[:, None]

    grid_n = (N_NODES // _ROWS,)

    h0, g0 = pl.pallas_call(
        _prep_gcn_body,
        grid=grid_n,
        in_specs=[_rows(HID), _rows(1), _full((HID, 2 * HID)),
                  _full((2 * HID,)), _full((2 * HID, 2 * HID))],
        out_specs=[_rows(2 * HID), _rows(2 * HID)],
        out_shape=[jax.ShapeDtypeStruct((N_NODES, 2 * HID), jnp.float32)] * 2,
    )(str_init, dinv, W_emb, b_emb, W_gcn0)

    acc0 = jax.ops.segment_sum(g0[src], dst, num_segments=N_NODES)

    h1, g1 = pl.pallas_call(
        _gcn_step_body,
        grid=grid_n,
        in_specs=[_rows(2 * HID), _rows(2 * HID), _rows(1),
                  _full((2 * HID,)), _full((2 * HID, 2 * HID))],
        out_specs=[_rows(2 * HID), _rows(2 * HID)],
        out_shape=[jax.ShapeDtypeStruct((N_NODES, 2 * HID), jnp.float32)] * 2,
    )(acc0, h0, dinv, b_gcn0, W_gcn1)

    acc1 = jax.ops.segment_sum(g1[src], dst, num_segments=N_NODES)

    str_out = pl.pallas_call(
        _gcn_final_body,
        grid=grid_n,
        in_specs=[_rows(2 * HID), _rows(2 * HID), _rows(1),
                  _full((2 * HID,)), _full((2 * HID, PRED)), _full((PRED,))],
        out_specs=_rows(PRED),
        out_shape=jax.ShapeDtypeStruct((N_NODES, PRED), jnp.float32),
    )(acc1, h1, dinv, b_gcn1, W_out, b_out)

    BN = BATCH * N_NODES
    tep_bn = tep_out.reshape(BN, HID)
    eyeH = jnp.eye(HEADS, dtype=jnp.float32)
    R = jnp.repeat(eyeH, PRED, axis=1)
    Asrc = jnp.repeat(eyeH, PRED, axis=0) * att_src.reshape(HF, 1)
    Adst = jnp.repeat(eyeH, PRED, axis=0) * att_dst.reshape(HF, 1)
    M = jnp.tile(jnp.eye(PRED, dtype=jnp.float32), (HEADS, 1))

    grid_bn = (BN // _ROWS,)
    h36, exs, a_s, a_d = pl.pallas_call(
        _gat_prep_body,
        grid=grid_bn,
        in_specs=[_rows(HID), _full((HID, HF)), _full((HF, HEADS)),
                  _full((HF, HEADS))],
        out_specs=[_rows(HF), _rows(HEADS), _rows(HEADS), _rows(HEADS)],
        out_shape=[jax.ShapeDtypeStruct((BN, HF), jnp.float32),
                   jax.ShapeDtypeStruct((BN, HEADS), jnp.float32),
                   jax.ShapeDtypeStruct((BN, HEADS), jnp.float32),
                   jax.ShapeDtypeStruct((BN, HEADS), jnp.float32)],
    )(tep_bn, W_gat, Asrc, Adst)

    z = a_s[src] + a_d[dst]
    ex = jnp.exp(jnp.where(z > 0, z, 0.2 * z))
    den = jax.ops.segment_sum(ex, dst, num_segments=N_NODES)
    ex36e = jnp.repeat(ex, PRED, axis=1)
    num = jax.ops.segment_sum(h36[src] * ex36e, dst, num_segments=N_NODES)

    spa = pl.pallas_call(
        _gat_combine_body,
        grid=(BN // _ROWS,),
        in_specs=[_rows(HF), _rows(HEADS),
                  _rows_clamped(HF, N_NODES // _ROWS),
                  _rows_clamped(HEADS, N_NODES // _ROWS),
                  _full((HEADS, HF)), _full((HF, PRED)), _full((PRED,))],
        out_specs=_rows(PRED),
        out_shape=jax.ShapeDtypeStruct((BN, PRED), jnp.float32),
    )(h36, exs, num, den, R, M, b_gat)

    str_emb = jnp.broadcast_to(str_out[None], (BATCH, N_NODES, PRED))
    return str_emb, spa.reshape(BATCH, N_NODES, PRED)


def kernel(x, str_init, edge_index, tep_out, W_emb, b_emb, W_gcn0, b_gcn0,
           W_gcn1, b_gcn1, W_out, b_out, W_gat, att_src, att_dst, b_gat):
    return _run(x, str_init, edge_index, tep_out, W_emb, b_emb, W_gcn0,
                b_gcn0, W_gcn1, b_gcn1, W_out, b_out, W_gat, att_src,
                att_dst, b_gat)


# speedup vs baseline: 12.3379x; 1.5228x over previous
import functools

import jax
import jax.numpy as jnp
from jax import lax
from jax.experimental import pallas as pl
from jax.experimental.pallas import tpu as pltpu
from jax.experimental.pallas import tpu_sc as plsc

N_NODES = 10000
N_EDGES = 320000
BATCH = 4
HID = 128
PRED = 12
HEADS = 3
HF = HEADS * PRED

_ROWS = 1000

_NSC = 2
_NTILE = 16
_EPB = 128
_EPAD = 327680
_EROWS = _EPAD // _EPB
_TPB = _EROWS // _NTILE
_SLAB = 32
_DPB = _EROWS // (_NSC * _NTILE)
_NROWS = 10240
_RPT = _NROWS // _NTILE
_TRASH = 10100

_sc_mesh = plsc.VectorSubcoreMesh(core_axis_name="c", subcore_axis_name="s")


def _sc_deg_body(dstp, zeros1, ones_h, out, idx_v, ones_v, acc_sh):
    c = lax.axis_index("c")
    s = lax.axis_index("s")
    w = c * _NTILE + s
    pltpu.sync_copy(zeros1.at[pl.ds(s * _RPT, _RPT)],
                    acc_sh.at[pl.ds(s * _RPT, _RPT)])
    pltpu.sync_copy(ones_h, ones_v)
    pltpu.sync_copy(dstp.at[pl.ds(w * _DPB, _DPB)], idx_v)
    plsc.subcore_barrier()

    def body(j, carry):
        pltpu.sync_copy(ones_v, acc_sh.at[idx_v.at[j]], add=True)
        return carry

    lax.fori_loop(0, _DPB, body, 0)
    plsc.subcore_barrier()
    pltpu.sync_copy(acc_sh.at[pl.ds(s * _RPT, _RPT)],
                    out.at[pl.ds(c * _NROWS + s * _RPT, _RPT)])


def _sc_scatter_body(g_hbm, srcp2, dstp, zeros2, out, idx_s, idx_d, rows,
                     acc_sh, sem):
    c = lax.axis_index("c")
    s = lax.axis_index("s")
    pltpu.sync_copy(zeros2.at[pl.ds(s * _RPT, _RPT)],
                    acc_sh.at[pl.ds(s * _RPT, _RPT)])
    plsc.subcore_barrier()

    def slab(t, carry):
        start = s * _TPB + t * _SLAB
        pltpu.sync_copy(srcp2.at[c, pl.ds(start, _SLAB)], idx_s)
        pltpu.sync_copy(dstp.at[pl.ds(start, _SLAB)], idx_d)

        def body(j, carry2):
            pltpu.async_copy(g_hbm.at[idx_s.at[j]], rows, sem).wait()
            pltpu.sync_copy(rows, acc_sh.at[idx_d.at[j]], add=True)
            return carry2

        lax.fori_loop(0, _SLAB, body, 0)
        return carry

    lax.fori_loop(0, _TPB // _SLAB, slab, 0)
    plsc.subcore_barrier()
    pltpu.sync_copy(acc_sh.at[pl.ds(s * _RPT, _RPT)],
                    out.at[c, pl.ds(s * _RPT, _RPT)])


_sc_deg = pl.kernel(
    _sc_deg_body,
    out_type=jax.ShapeDtypeStruct((_NSC * _NROWS,), jnp.float32),
    mesh=_sc_mesh,
    scratch_types=[
        pltpu.VMEM((_DPB, _EPB), jnp.int32),
        pltpu.VMEM((_EPB,), jnp.float32),
        pltpu.VMEM_SHARED((_NROWS,), jnp.float32),
    ],
)

_sc_scatter = pl.kernel(
    _sc_scatter_body,
    out_type=jax.ShapeDtypeStruct((_NSC, _NROWS, HID), jnp.float32),
    mesh=_sc_mesh,
    scratch_types=[
        pltpu.VMEM((_SLAB, _EPB), jnp.int32),
        pltpu.VMEM((_SLAB, _EPB), jnp.int32),
        pltpu.VMEM((_EPB, HID), jnp.float32),
        pltpu.VMEM_SHARED((_NROWS, HID), jnp.float32),
        pltpu.SemaphoreType.DMA,
    ],
)


def _prep_gcn_body(x_ref, dinv_ref, We_ref, be_ref, W0_ref, h0_ref, g0_ref):
    s0 = jnp.dot(x_ref[...], We_ref[...], preferred_element_type=jnp.float32)
    s0 = s0 + be_ref[...]
    h0 = jnp.dot(s0, W0_ref[...], preferred_element_type=jnp.float32)
    h0_ref[...] = h0
    g = h0 * dinv_ref[...]
    g0_ref[0] = g[:, :HID]
    g0_ref[1] = g[:, HID:]


def _gcn_step_body(acc_ref, h_ref, dinv_ref, b_ref, Wn_ref, hn_ref, gn_ref):
    dinv = dinv_ref[...]
    acc = jnp.concatenate([acc_ref[0], acc_ref[1]], axis=1)
    s = jnp.tanh(dinv * acc + dinv * dinv * h_ref[...] + b_ref[...])
    hn = jnp.dot(s, Wn_ref[...], preferred_element_type=jnp.float32)
    hn_ref[...] = hn
    g = hn * dinv
    gn_ref[0] = g[:, :HID]
    gn_ref[1] = g[:, HID:]


def _gcn_final_body(acc_ref, h_ref, dinv_ref, b_ref, Wo_ref, bo_ref, out_ref):
    dinv = dinv_ref[...]
    acc = jnp.concatenate([acc_ref[0], acc_ref[1]], axis=1)
    s = jnp.tanh(dinv * acc + dinv * dinv * h_ref[...] + b_ref[...])
    out_ref[...] = jnp.dot(s, Wo_ref[...], preferred_element_type=jnp.float32) + bo_ref[...]


def _gat_prep_body(x_ref, Wg_ref, As_ref, Ad_ref, h_ref, exs_ref, asrc_ref, adst_ref):
    h = jnp.dot(x_ref[...], Wg_ref[...], preferred_element_type=jnp.float32)
    h_ref[...] = h
    a_s = jnp.dot(h, As_ref[...], preferred_element_type=jnp.float32)
    a_d = jnp.dot(h, Ad_ref[...], preferred_element_type=jnp.float32)
    asrc_ref[...] = a_s
    adst_ref[...] = a_d
    z = a_s + a_d
    exs_ref[...] = jnp.exp(jnp.where(z > 0, z, 0.2 * z))


def _gat_combine_body(h_ref, exs_ref, num_ref, den_ref, R_ref, M_ref, b_ref,
                      out_ref):
    i = pl.program_id(0)
    real = (i < N_NODES // _ROWS).astype(jnp.float32)
    h = h_ref[...]
    exs3 = exs_ref[...]
    ex36 = jnp.dot(exs3, R_ref[...], preferred_element_type=jnp.float32)
    num = num_ref[...] * real + h * ex36
    den3 = den_ref[...] * real + exs3
    den36 = jnp.dot(den3, R_ref[...], preferred_element_type=jnp.float32) + 1e-16
    ratio = num / den36
    out_ref[...] = (
        jnp.dot(ratio, M_ref[...], preferred_element_type=jnp.float32) / HEADS
        + b_ref[...]
    )


def _full(shape):
    return pl.BlockSpec(shape, lambda i: (0,) * len(shape))


def _rows(width, blk=_ROWS):
    return pl.BlockSpec((blk, width), lambda i: (i, 0))


def _rows_clamped(width, nblk, blk=_ROWS):
    return pl.BlockSpec((blk, width), lambda i: (jnp.minimum(i, nblk - 1), 0))


@jax.jit
def _run(x, str_init, edge_index, tep_out, W_emb, b_emb, W_gcn0, b_gcn0,
         W_gcn1, b_gcn1, W_out, b_out, W_gat, att_src, att_dst, b_gat):
    src = edge_index[0].astype(jnp.int32)
    dst = edge_index[1].astype(jnp.int32)

    pad = _EPAD - N_EDGES
    srcp = jnp.concatenate([src, jnp.zeros((pad,), jnp.int32)])
    srcp2 = jnp.stack([srcp, srcp + N_NODES]).reshape(_NSC, _EROWS, _EPB)
    dstp = jnp.concatenate(
        [dst, jnp.full((pad,), _TRASH, jnp.int32)]).reshape(_EROWS, _EPB)
    zeros1 = jnp.zeros((_NROWS,), jnp.float32)
    zeros2 = jnp.zeros((_NROWS, HID), jnp.float32)
    ones_h = jnp.ones((_EPB,), jnp.float32)

    deg_parts = _sc_deg(dstp, zeros1, ones_h)
    dinv = jax.lax.rsqrt(
        deg_parts[:N_NODES] + deg_parts[_NROWS:_NROWS + N_NODES] + 1.0)[:, None]

    grid_n = (N_NODES // _ROWS,)
    _g_spec = pl.BlockSpec((_NSC, _ROWS, HID), lambda i: (0, i, 0))
    _acc_spec = pl.BlockSpec((_NSC, _ROWS, HID), lambda i: (0, i, 0))
    _g_shape = jax.ShapeDtypeStruct((_NSC, N_NODES, HID), jnp.float32)

    h0, g0 = pl.pallas_call(
        _prep_gcn_body,
        grid=grid_n,
        in_specs=[_rows(HID), _rows(1), _full((HID, 2 * HID)),
                  _full((2 * HID,)), _full((2 * HID, 2 * HID))],
        out_specs=[_rows(2 * HID), _g_spec],
        out_shape=[jax.ShapeDtypeStruct((N_NODES, 2 * HID), jnp.float32),
                   _g_shape],
    )(str_init, dinv, W_emb, b_emb, W_gcn0)

    acc0 = _sc_scatter(g0.reshape(_NSC * N_NODES, HID), srcp2, dstp, zeros2)

    h1, g1 = pl.pallas_call(
        _gcn_step_body,
        grid=grid_n,
        in_specs=[_acc_spec, _rows(2 * HID), _rows(1),
                  _full((2 * HID,)), _full((2 * HID, 2 * HID))],
        out_specs=[_rows(2 * HID), _g_spec],
        out_shape=[jax.ShapeDtypeStruct((N_NODES, 2 * HID), jnp.float32),
                   _g_shape],
    )(acc0, h0, dinv, b_gcn0, W_gcn1)

    acc1 = _sc_scatter(g1.reshape(_NSC * N_NODES, HID), srcp2, dstp, zeros2)

    str_out = pl.pallas_call(
        _gcn_final_body,
        grid=grid_n,
        in_specs=[_acc_spec, _rows(2 * HID), _rows(1),
                  _full((2 * HID,)), _full((2 * HID, PRED)), _full((PRED,))],
        out_specs=_rows(PRED),
        out_shape=jax.ShapeDtypeStruct((N_NODES, PRED), jnp.float32),
    )(acc1, h1, dinv, b_gcn1, W_out, b_out)

    BN = BATCH * N_NODES
    tep_bn = tep_out.reshape(BN, HID)
    eyeH = jnp.eye(HEADS, dtype=jnp.float32)
    R = jnp.repeat(eyeH, PRED, axis=1)
    Asrc = jnp.repeat(eyeH, PRED, axis=0) * att_src.reshape(HF, 1)
    Adst = jnp.repeat(eyeH, PRED, axis=0) * att_dst.reshape(HF, 1)
    M = jnp.tile(jnp.eye(PRED, dtype=jnp.float32), (HEADS, 1))

    grid_bn = (BN // _ROWS,)
    h36, exs, a_s, a_d = pl.pallas_call(
        _gat_prep_body,
        grid=grid_bn,
        in_specs=[_rows(HID), _full((HID, HF)), _full((HF, HEADS)),
                  _full((HF, HEADS))],
        out_specs=[_rows(HF), _rows(HEADS), _rows(HEADS), _rows(HEADS)],
        out_shape=[jax.ShapeDtypeStruct((BN, HF), jnp.float32),
                   jax.ShapeDtypeStruct((BN, HEADS), jnp.float32),
                   jax.ShapeDtypeStruct((BN, HEADS), jnp.float32),
                   jax.ShapeDtypeStruct((BN, HEADS), jnp.float32)],
    )(tep_bn, W_gat, Asrc, Adst)

    z = a_s[src] + a_d[dst]
    ex = jnp.exp(jnp.where(z > 0, z, 0.2 * z))
    den = jax.ops.segment_sum(ex, dst, num_segments=N_NODES)
    ex36e = jnp.repeat(ex, PRED, axis=1)
    num = jax.ops.segment_sum(h36[src] * ex36e, dst, num_segments=N_NODES)

    spa = pl.pallas_call(
        _gat_combine_body,
        grid=(BN // _ROWS,),
        in_specs=[_rows(HF), _rows(HEADS),
                  _rows_clamped(HF, N_NODES // _ROWS),
                  _rows_clamped(HEADS, N_NODES // _ROWS),
                  _full((HEADS, HF)), _full((HF, PRED)), _full((PRED,))],
        out_specs=_rows(PRED),
        out_shape=jax.ShapeDtypeStruct((BN, PRED), jnp.float32),
    )(h36, exs, num, den, R, M, b_gat)

    str_emb = jnp.broadcast_to(str_out[None], (BATCH, N_NODES, PRED))
    return str_emb, spa.reshape(BATCH, N_NODES, PRED)


def kernel(x, str_init, edge_index, tep_out, W_emb, b_emb, W_gcn0, b_gcn0,
           W_gcn1, b_gcn1, W_out, b_out, W_gat, att_src, att_dst, b_gat):
    return _run(x, str_init, edge_index, tep_out, W_emb, b_emb, W_gcn0,
                b_gcn0, W_gcn1, b_gcn1, W_out, b_out, W_gat, att_src,
                att_dst, b_gat)


# speedup vs baseline: 24.3787x; 1.9759x over previous
import functools

import jax
import jax.numpy as jnp
from jax import lax
from jax.experimental import pallas as pl
from jax.experimental.pallas import tpu as pltpu
from jax.experimental.pallas import tpu_sc as plsc

N_NODES = 10000
N_EDGES = 320000
BATCH = 4
HID = 128
PRED = 12
HEADS = 3
HF = HEADS * PRED

_ROWS = 1000

_NSC = 2
_NTILE = 16
_EPB = 128
_EPAD = 327680
_EROWS = _EPAD // _EPB
_TPB = _EROWS // _NTILE
_SLAB = 32
_DPB = _EROWS // (_NSC * _NTILE)
_NROWS = 10240
_RPT = _NROWS // _NTILE
_TRASH = 10100

_sc_mesh = plsc.VectorSubcoreMesh(core_axis_name="c", subcore_axis_name="s")


def _sc_deg_body(dstp, zeros1, ones_h, out, idx_v, ones_v, acc_sh):
    c = lax.axis_index("c")
    s = lax.axis_index("s")
    w = c * _NTILE + s
    pltpu.sync_copy(zeros1.at[pl.ds(s * _RPT, _RPT)],
                    acc_sh.at[pl.ds(s * _RPT, _RPT)])
    pltpu.sync_copy(ones_h, ones_v)
    pltpu.sync_copy(dstp.at[pl.ds(w * _DPB, _DPB)], idx_v)
    plsc.subcore_barrier()

    def body(j, carry):
        pltpu.sync_copy(ones_v, acc_sh.at[idx_v.at[j]], add=True)
        return carry

    lax.fori_loop(0, _DPB, body, 0)
    plsc.subcore_barrier()
    pltpu.sync_copy(acc_sh.at[pl.ds(s * _RPT, _RPT)],
                    out.at[pl.ds(c * _NROWS + s * _RPT, _RPT)])


def _sc_scatter_body(g_hbm, srcp2, dstp, zeros2, out, idx_s, idx_d, rows,
                     acc_sh, sem):
    c = lax.axis_index("c")
    s = lax.axis_index("s")
    pltpu.sync_copy(zeros2.at[pl.ds(s * _RPT, _RPT)],
                    acc_sh.at[pl.ds(s * _RPT, _RPT)])
    plsc.subcore_barrier()

    def slab(t, carry):
        start = s * _TPB + t * _SLAB
        pltpu.sync_copy(srcp2.at[c, pl.ds(start, _SLAB)], idx_s)
        pltpu.sync_copy(dstp.at[pl.ds(start, _SLAB)], idx_d)

        def body(j, carry2):
            pltpu.async_copy(g_hbm.at[idx_s.at[j]], rows, sem).wait()
            pltpu.sync_copy(rows, acc_sh.at[idx_d.at[j]], add=True)
            return carry2

        lax.fori_loop(0, _SLAB, body, 0)
        return carry

    lax.fori_loop(0, _TPB // _SLAB, slab, 0)
    plsc.subcore_barrier()
    pltpu.sync_copy(acc_sh.at[pl.ds(s * _RPT, _RPT)],
                    out.at[c, pl.ds(s * _RPT, _RPT)])


_HP = 16
_HF48 = HEADS * _HP


def _sc_gat_body(h48_hbm, asd_hbm, srcp_r, dstp, zeros48, zeros1,
                 num_out, den_out,
                 asd_v, idx_s_v, idx_d_v, rows, exb, num_sh,
                 den0_sh, den1_sh, den2_sh, sem):
    c = lax.axis_index("c")
    s = lax.axis_index("s")
    w = c * _NTILE + s
    dens = (den0_sh, den1_sh, den2_sh)
    pltpu.sync_copy(zeros48.at[pl.ds(s * _RPT, _RPT)],
                    num_sh.at[pl.ds(s * _RPT, _RPT)])
    for h in range(HEADS):
        pltpu.sync_copy(zeros1.at[pl.ds(s * _RPT, _RPT)],
                        dens[h].at[pl.ds(s * _RPT, _RPT)])
    pltpu.sync_copy(srcp_r.at[pl.ds(w * _DPB, _DPB)], idx_s_v)
    pltpu.sync_copy(dstp.at[pl.ds(w * _DPB, _DPB)], idx_d_v)
    pltpu.sync_copy(asd_hbm, asd_v)
    plsc.subcore_barrier()

    def row_body(j, carry):
        pltpu.async_copy(h48_hbm.at[idx_s_v.at[j]], rows, sem).wait()
        for k in range(_EPB // 16):
            si = idx_s_v[j, pl.ds(16 * k, 16)]
            di = idx_d_v[j, pl.ds(16 * k, 16)]
            for h in range(HEADS):
                a1 = plsc.load_gather(asd_v, [si + (h * _NROWS)])
                a2 = plsc.load_gather(asd_v, [di + ((HEADS + h) * _NROWS)])
                z = a1 + a2
                z = jnp.where(z > 0, z, 0.2 * z)
                exb[pl.ds(h * _EPB + 16 * k, 16)] = jnp.exp(z)
        for e in range(_EPB):
            for h in range(HEADS):
                sp = plsc.load_gather(
                    exb, [jnp.full((16,), h * _EPB + e, jnp.int32)])
                rows[e, pl.ds(_HP * h, _HP)] = rows[e, pl.ds(_HP * h, _HP)] * sp
        pltpu.sync_copy(rows, num_sh.at[idx_d_v.at[j]], add=True)
        for h in range(HEADS):
            pltpu.sync_copy(exb.at[pl.ds(h * _EPB, _EPB)],
                            dens[h].at[idx_d_v.at[j]], add=True)
        return carry

    lax.fori_loop(0, _DPB, row_body, 0)
    plsc.subcore_barrier()
    pltpu.sync_copy(num_sh.at[pl.ds(s * _RPT, _RPT)],
                    num_out.at[c, pl.ds(s * _RPT, _RPT)])
    for h in range(HEADS):
        pltpu.sync_copy(
            dens[h].at[pl.ds(s * _RPT, _RPT)],
            den_out.at[pl.ds(c * HEADS * _NROWS + h * _NROWS + s * _RPT,
                             _RPT)])


_sc_gat = pl.kernel(
    _sc_gat_body,
    out_type=(jax.ShapeDtypeStruct((_NSC, _NROWS, _HF48), jnp.float32),
              jax.ShapeDtypeStruct((_NSC * HEADS * _NROWS,), jnp.float32)),
    mesh=_sc_mesh,
    compiler_params=pltpu.CompilerParams(needs_layout_passes=False,
                                         use_tc_tiling_on_sc=False),
    scratch_types=[
        pltpu.VMEM((2 * HEADS * _NROWS,), jnp.float32),
        pltpu.VMEM((_DPB, _EPB), jnp.int32),
        pltpu.VMEM((_DPB, _EPB), jnp.int32),
        pltpu.VMEM((_EPB, _HF48), jnp.float32),
        pltpu.VMEM((HEADS * _EPB,), jnp.float32),
        pltpu.VMEM_SHARED((_NROWS, _HF48), jnp.float32),
        pltpu.VMEM_SHARED((_NROWS,), jnp.float32),
        pltpu.VMEM_SHARED((_NROWS,), jnp.float32),
        pltpu.VMEM_SHARED((_NROWS,), jnp.float32),
        pltpu.SemaphoreType.DMA,
    ],
)


_sc_deg = pl.kernel(
    _sc_deg_body,
    out_type=jax.ShapeDtypeStruct((_NSC * _NROWS,), jnp.float32),
    mesh=_sc_mesh,
    compiler_params=pltpu.CompilerParams(needs_layout_passes=False),
    scratch_types=[
        pltpu.VMEM((_DPB, _EPB), jnp.int32),
        pltpu.VMEM((_EPB,), jnp.float32),
        pltpu.VMEM_SHARED((_NROWS,), jnp.float32),
    ],
)

_sc_scatter = pl.kernel(
    _sc_scatter_body,
    out_type=jax.ShapeDtypeStruct((_NSC, _NROWS, HID), jnp.float32),
    mesh=_sc_mesh,
    compiler_params=pltpu.CompilerParams(needs_layout_passes=False),
    scratch_types=[
        pltpu.VMEM((_SLAB, _EPB), jnp.int32),
        pltpu.VMEM((_SLAB, _EPB), jnp.int32),
        pltpu.VMEM((_EPB, HID), jnp.float32),
        pltpu.VMEM_SHARED((_NROWS, HID), jnp.float32),
        pltpu.SemaphoreType.DMA,
    ],
)


def _prep_gcn_body(x_ref, dinv_ref, We_ref, be_ref, W0_ref, h0_ref, g0_ref):
    s0 = jnp.dot(x_ref[...], We_ref[...], preferred_element_type=jnp.float32)
    s0 = s0 + be_ref[...]
    h0 = jnp.dot(s0, W0_ref[...], preferred_element_type=jnp.float32)
    h0_ref[...] = h0
    g = h0 * dinv_ref[...]
    g0_ref[0] = g[:, :HID]
    g0_ref[1] = g[:, HID:]


def _gcn_step_body(acc_ref, h_ref, dinv_ref, b_ref, Wn_ref, hn_ref, gn_ref):
    dinv = dinv_ref[...]
    acc = jnp.concatenate([acc_ref[0], acc_ref[1]], axis=1)
    s = jnp.tanh(dinv * acc + dinv * dinv * h_ref[...] + b_ref[...])
    hn = jnp.dot(s, Wn_ref[...], preferred_element_type=jnp.float32)
    hn_ref[...] = hn
    g = hn * dinv
    gn_ref[0] = g[:, :HID]
    gn_ref[1] = g[:, HID:]


def _gcn_final_body(acc_ref, h_ref, dinv_ref, b_ref, Wo_ref, bo_ref, out_ref):
    dinv = dinv_ref[...]
    acc = jnp.concatenate([acc_ref[0], acc_ref[1]], axis=1)
    s = jnp.tanh(dinv * acc + dinv * dinv * h_ref[...] + b_ref[...])
    out_ref[...] = jnp.dot(s, Wo_ref[...], preferred_element_type=jnp.float32) + bo_ref[...]


def _gat_prep_body(x_ref, Wg_ref, As_ref, Ad_ref, h_ref, exs_ref, asrc_ref, adst_ref):
    h = jnp.dot(x_ref[...], Wg_ref[...], preferred_element_type=jnp.float32)
    h_ref[...] = h
    a_s = jnp.dot(h, As_ref[...], preferred_element_type=jnp.float32)
    a_d = jnp.dot(h, Ad_ref[...], preferred_element_type=jnp.float32)
    asrc_ref[...] = a_s
    adst_ref[...] = a_d
    z = a_s + a_d
    exs_ref[...] = jnp.exp(jnp.where(z > 0, z, 0.2 * z))


def _gat_combine_body(h_ref, exs_ref, num_ref, den_ref, R_ref, M_ref, b_ref,
                      out_ref):
    i = pl.program_id(0)
    real = (i < N_NODES // _ROWS).astype(jnp.float32)
    h = h_ref[...]
    exs3 = exs_ref[...]
    ex48 = jnp.dot(exs3, R_ref[...], preferred_element_type=jnp.float32)
    num = (num_ref[0] + num_ref[1]) * real + h * ex48
    den3 = den_ref[...] * real + exs3
    den48 = jnp.dot(den3, R_ref[...], preferred_element_type=jnp.float32) + 1e-16
    ratio = num / den48
    out_ref[...] = (
        jnp.dot(ratio, M_ref[...], preferred_element_type=jnp.float32) / HEADS
        + b_ref[...]
    )


def _full(shape):
    return pl.BlockSpec(shape, lambda i: (0,) * len(shape))


def _rows(width, blk=_ROWS):
    return pl.BlockSpec((blk, width), lambda i: (i, 0))


def _rows_clamped(width, nblk, blk=_ROWS):
    return pl.BlockSpec((blk, width), lambda i: (jnp.minimum(i, nblk - 1), 0))


@jax.jit
def _run(x, str_init, edge_index, tep_out, W_emb, b_emb, W_gcn0, b_gcn0,
         W_gcn1, b_gcn1, W_out, b_out, W_gat, att_src, att_dst, b_gat):
    src = edge_index[0].astype(jnp.int32)
    dst = edge_index[1].astype(jnp.int32)

    pad = _EPAD - N_EDGES
    srcp = jnp.concatenate([src, jnp.zeros((pad,), jnp.int32)])
    srcp2 = jnp.stack([srcp, srcp + N_NODES]).reshape(_NSC, _EROWS, _EPB)
    dstp = jnp.concatenate(
        [dst, jnp.full((pad,), _TRASH, jnp.int32)]).reshape(_EROWS, _EPB)
    zeros1 = jnp.zeros((_NROWS,), jnp.float32)
    zeros2 = jnp.zeros((_NROWS, HID), jnp.float32)
    ones_h = jnp.ones((_EPB,), jnp.float32)

    deg_parts = _sc_deg(dstp, zeros1, ones_h)
    dinv = jax.lax.rsqrt(
        deg_parts[:N_NODES] + deg_parts[_NROWS:_NROWS + N_NODES] + 1.0)[:, None]

    grid_n = (N_NODES // _ROWS,)
    _g_spec = pl.BlockSpec((_NSC, _ROWS, HID), lambda i: (0, i, 0))
    _acc_spec = pl.BlockSpec((_NSC, _ROWS, HID), lambda i: (0, i, 0))
    _g_shape = jax.ShapeDtypeStruct((_NSC, N_NODES, HID), jnp.float32)

    h0, g0 = pl.pallas_call(
        _prep_gcn_body,
        grid=grid_n,
        in_specs=[_rows(HID), _rows(1), _full((HID, 2 * HID)),
                  _full((2 * HID,)), _full((2 * HID, 2 * HID))],
        out_specs=[_rows(2 * HID), _g_spec],
        out_shape=[jax.ShapeDtypeStruct((N_NODES, 2 * HID), jnp.float32),
                   _g_shape],
    )(str_init, dinv, W_emb, b_emb, W_gcn0)

    acc0 = _sc_scatter(g0.reshape(_NSC * N_NODES, HID), srcp2, dstp, zeros2)

    h1, g1 = pl.pallas_call(
        _gcn_step_body,
        grid=grid_n,
        in_specs=[_acc_spec, _rows(2 * HID), _rows(1),
                  _full((2 * HID,)), _full((2 * HID, 2 * HID))],
        out_specs=[_rows(2 * HID), _g_spec],
        out_shape=[jax.ShapeDtypeStruct((N_NODES, 2 * HID), jnp.float32),
                   _g_shape],
    )(acc0, h0, dinv, b_gcn0, W_gcn1)

    acc1 = _sc_scatter(g1.reshape(_NSC * N_NODES, HID), srcp2, dstp, zeros2)

    str_out = pl.pallas_call(
        _gcn_final_body,
        grid=grid_n,
        in_specs=[_acc_spec, _rows(2 * HID), _rows(1),
                  _full((2 * HID,)), _full((2 * HID, PRED)), _full((PRED,))],
        out_specs=_rows(PRED),
        out_shape=jax.ShapeDtypeStruct((N_NODES, PRED), jnp.float32),
    )(acc1, h1, dinv, b_gcn1, W_out, b_out)

    BN = BATCH * N_NODES
    tep_bn = tep_out.reshape(BN, HID)
    eyeH = jnp.eye(HEADS, dtype=jnp.float32)
    R = jnp.repeat(eyeH, _HP, axis=1)
    att_s48 = jnp.pad(att_src, ((0, 0), (0, _HP - PRED))).reshape(_HF48)
    att_d48 = jnp.pad(att_dst, ((0, 0), (0, _HP - PRED))).reshape(_HF48)
    Asrc = jnp.repeat(eyeH, _HP, axis=0) * att_s48[:, None]
    Adst = jnp.repeat(eyeH, _HP, axis=0) * att_d48[:, None]
    M = jnp.tile(jnp.pad(jnp.eye(PRED, dtype=jnp.float32),
                         ((0, _HP - PRED), (0, 0))), (HEADS, 1))
    W48 = jnp.pad(W_gat.reshape(HID, HEADS, PRED),
                  ((0, 0), (0, 0), (0, _HP - PRED))).reshape(HID, _HF48)

    grid_bn = (BN // _ROWS,)
    h48, exs, a_s, a_d = pl.pallas_call(
        _gat_prep_body,
        grid=grid_bn,
        in_specs=[_rows(HID), _full((HID, _HF48)), _full((_HF48, HEADS)),
                  _full((_HF48, HEADS))],
        out_specs=[_rows(_HF48), _rows(HEADS), _rows(HEADS), _rows(HEADS)],
        out_shape=[jax.ShapeDtypeStruct((BN, _HF48), jnp.float32),
                   jax.ShapeDtypeStruct((BN, HEADS), jnp.float32),
                   jax.ShapeDtypeStruct((BN, HEADS), jnp.float32),
                   jax.ShapeDtypeStruct((BN, HEADS), jnp.float32)],
    )(tep_bn, W48, Asrc, Adst)

    asd = jnp.pad(
        jnp.concatenate([a_s[:N_NODES].T, a_d[:N_NODES].T]),
        ((0, 0), (0, _NROWS - N_NODES))).reshape(-1)
    zeros48 = jnp.zeros((_NROWS, _HF48), jnp.float32)
    num_parts, den_flat = _sc_gat(h48, asd, srcp2[0], dstp, zeros48, zeros1)
    den_p = den_flat.reshape(_NSC, HEADS, _NROWS)
    den3 = (den_p[0] + den_p[1]).T

    spa = pl.pallas_call(
        _gat_combine_body,
        grid=(BN // _ROWS,),
        in_specs=[_rows(_HF48), _rows(HEADS),
                  pl.BlockSpec((_NSC, _ROWS, _HF48),
                               lambda i: (0, jnp.minimum(i, N_NODES // _ROWS - 1), 0)),
                  _rows_clamped(HEADS, N_NODES // _ROWS),
                  _full((HEADS, _HF48)), _full((_HF48, PRED)), _full((PRED,))],
        out_specs=_rows(PRED),
        out_shape=jax.ShapeDtypeStruct((BN, PRED), jnp.float32),
    )(h48, exs, num_parts, den3, R, M, b_gat)

    str_emb = jnp.broadcast_to(str_out[None], (BATCH, N_NODES, PRED))
    return str_emb, spa.reshape(BATCH, N_NODES, PRED)


def kernel(x, str_init, edge_index, tep_out, W_emb, b_emb, W_gcn0, b_gcn0,
           W_gcn1, b_gcn1, W_out, b_out, W_gat, att_src, att_dst, b_gat):
    return _run(x, str_init, edge_index, tep_out, W_emb, b_emb, W_gcn0,
                b_gcn0, W_gcn1, b_gcn1, W_out, b_out, W_gat, att_src,
                att_dst, b_gat)


# speedup vs baseline: 26.5224x; 1.0879x over previous
import functools

import jax
import jax.numpy as jnp
from jax import lax
from jax.experimental import pallas as pl
from jax.experimental.pallas import tpu as pltpu
from jax.experimental.pallas import tpu_sc as plsc

N_NODES = 10000
N_EDGES = 320000
BATCH = 4
HID = 128
PRED = 12
HEADS = 3
HF = HEADS * PRED

_ROWS = 1000

_NSC = 2
_NTILE = 16
_EPB = 128
_EPAD = 327680
_EROWS = _EPAD // _EPB
_TPB = _EROWS // _NTILE
_SLAB = 32
_DPB = _EROWS // (_NSC * _NTILE)
_NROWS = 10240
_RPT = _NROWS // _NTILE
_TRASH = 10100

_sc_mesh = plsc.VectorSubcoreMesh(core_axis_name="c", subcore_axis_name="s")


def _sc_deg_body(dstp, zeros1, ones_h, out, idx_v, ones_v, acc_sh):
    c = lax.axis_index("c")
    s = lax.axis_index("s")
    w = c * _NTILE + s
    pltpu.sync_copy(zeros1.at[pl.ds(s * _RPT, _RPT)],
                    acc_sh.at[pl.ds(s * _RPT, _RPT)])
    pltpu.sync_copy(ones_h, ones_v)
    pltpu.sync_copy(dstp.at[pl.ds(w * _DPB, _DPB)], idx_v)
    plsc.subcore_barrier()

    def body(j, carry):
        pltpu.sync_copy(ones_v, acc_sh.at[idx_v.at[j]], add=True)
        return carry

    lax.fori_loop(0, _DPB, body, 0)
    plsc.subcore_barrier()
    pltpu.sync_copy(acc_sh.at[pl.ds(s * _RPT, _RPT)],
                    out.at[pl.ds(c * _NROWS + s * _RPT, _RPT)])


def _sc_scatter_body(g_hbm, srcp2, dstp, zeros2, out, idx_s, idx_d, rows_a,
                     rows_b, acc_sh, sem_a, sem_b):
    c = lax.axis_index("c")
    s = lax.axis_index("s")
    pltpu.sync_copy(zeros2.at[pl.ds(s * _RPT, _RPT)],
                    acc_sh.at[pl.ds(s * _RPT, _RPT)])
    plsc.subcore_barrier()

    npair = _SLAB // 2

    def slab(t, carry):
        start = s * _TPB + t * _SLAB
        pltpu.sync_copy(srcp2.at[c, pl.ds(start, _SLAB)], idx_s)
        pltpu.sync_copy(dstp.at[pl.ds(start, _SLAB)], idx_d)
        pltpu.async_copy(g_hbm.at[idx_s.at[0]], rows_a, sem_a)

        def pair(j, carry2):
            r0 = 2 * j
            r1 = 2 * j + 1
            pltpu.make_async_copy(g_hbm.at[idx_s.at[r0]], rows_a, sem_a).wait()
            pltpu.async_copy(g_hbm.at[idx_s.at[r1]], rows_b, sem_b)
            pltpu.sync_copy(rows_a, acc_sh.at[idx_d.at[r0]], add=True)
            pltpu.make_async_copy(g_hbm.at[idx_s.at[r1]], rows_b, sem_b).wait()

            @pl.when(j < npair - 1)
            def _():
                pltpu.async_copy(g_hbm.at[idx_s.at[r1 + 1]], rows_a, sem_a)

            pltpu.sync_copy(rows_b, acc_sh.at[idx_d.at[r1]], add=True)
            return carry2

        lax.fori_loop(0, npair, pair, 0)
        return carry

    lax.fori_loop(0, _TPB // _SLAB, slab, 0)
    plsc.subcore_barrier()
    pltpu.sync_copy(acc_sh.at[pl.ds(s * _RPT, _RPT)],
                    out.at[c, pl.ds(s * _RPT, _RPT)])


_HP = 16
_HF48 = HEADS * _HP


def _sc_gat_body(h48_hbm, asd_hbm, srcp_r, dstp, zeros48, zeros1,
                 num_out, den_out,
                 asd_v, idx_s_v, idx_d_v, rows, exb, num_sh,
                 den0_sh, den1_sh, den2_sh, sem):
    c = lax.axis_index("c")
    s = lax.axis_index("s")
    w = c * _NTILE + s
    dens = (den0_sh, den1_sh, den2_sh)
    pltpu.sync_copy(zeros48.at[pl.ds(s * _RPT, _RPT)],
                    num_sh.at[pl.ds(s * _RPT, _RPT)])
    for h in range(HEADS):
        pltpu.sync_copy(zeros1.at[pl.ds(s * _RPT, _RPT)],
                        dens[h].at[pl.ds(s * _RPT, _RPT)])
    pltpu.sync_copy(srcp_r.at[pl.ds(w * _DPB, _DPB)], idx_s_v)
    pltpu.sync_copy(dstp.at[pl.ds(w * _DPB, _DPB)], idx_d_v)
    pltpu.sync_copy(asd_hbm, asd_v)
    plsc.subcore_barrier()

    def row_body(j, carry):
        pltpu.async_copy(h48_hbm.at[idx_s_v.at[j]], rows, sem).wait()
        for k in range(_EPB // 16):
            si = idx_s_v[j, pl.ds(16 * k, 16)]
            di = idx_d_v[j, pl.ds(16 * k, 16)]
            for h in range(HEADS):
                a1 = plsc.load_gather(asd_v, [si + (h * _NROWS)])
                a2 = plsc.load_gather(asd_v, [di + ((HEADS + h) * _NROWS)])
                z = a1 + a2
                z = jnp.where(z > 0, z, 0.2 * z)
                exb[pl.ds(h * _EPB + 16 * k, 16)] = jnp.exp(z)
        for e in range(_EPB):
            for h in range(HEADS):
                sp = plsc.load_gather(
                    exb, [jnp.full((16,), h * _EPB + e, jnp.int32)])
                rows[e, pl.ds(_HP * h, _HP)] = rows[e, pl.ds(_HP * h, _HP)] * sp
        pltpu.sync_copy(rows, num_sh.at[idx_d_v.at[j]], add=True)
        for h in range(HEADS):
            pltpu.sync_copy(exb.at[pl.ds(h * _EPB, _EPB)],
                            dens[h].at[idx_d_v.at[j]], add=True)
        return carry

    lax.fori_loop(0, _DPB, row_body, 0)
    plsc.subcore_barrier()
    pltpu.sync_copy(num_sh.at[pl.ds(s * _RPT, _RPT)],
                    num_out.at[c, pl.ds(s * _RPT, _RPT)])
    for h in range(HEADS):
        pltpu.sync_copy(
            dens[h].at[pl.ds(s * _RPT, _RPT)],
            den_out.at[pl.ds(c * HEADS * _NROWS + h * _NROWS + s * _RPT,
                             _RPT)])


_sc_gat = pl.kernel(
    _sc_gat_body,
    out_type=(jax.ShapeDtypeStruct((_NSC, _NROWS, _HF48), jnp.float32),
              jax.ShapeDtypeStruct((_NSC * HEADS * _NROWS,), jnp.float32)),
    mesh=_sc_mesh,
    compiler_params=pltpu.CompilerParams(needs_layout_passes=False,
                                         use_tc_tiling_on_sc=False),
    scratch_types=[
        pltpu.VMEM((2 * HEADS * _NROWS,), jnp.float32),
        pltpu.VMEM((_DPB, _EPB), jnp.int32),
        pltpu.VMEM((_DPB, _EPB), jnp.int32),
        pltpu.VMEM((_EPB, _HF48), jnp.float32),
        pltpu.VMEM((HEADS * _EPB,), jnp.float32),
        pltpu.VMEM_SHARED((_NROWS, _HF48), jnp.float32),
        pltpu.VMEM_SHARED((_NROWS,), jnp.float32),
        pltpu.VMEM_SHARED((_NROWS,), jnp.float32),
        pltpu.VMEM_SHARED((_NROWS,), jnp.float32),
        pltpu.SemaphoreType.DMA,
    ],
)


_sc_deg = pl.kernel(
    _sc_deg_body,
    out_type=jax.ShapeDtypeStruct((_NSC * _NROWS,), jnp.float32),
    mesh=_sc_mesh,
    compiler_params=pltpu.CompilerParams(needs_layout_passes=False),
    scratch_types=[
        pltpu.VMEM((_DPB, _EPB), jnp.int32),
        pltpu.VMEM((_EPB,), jnp.float32),
        pltpu.VMEM_SHARED((_NROWS,), jnp.float32),
    ],
)

_sc_scatter = pl.kernel(
    _sc_scatter_body,
    out_type=jax.ShapeDtypeStruct((_NSC, _NROWS, HID), jnp.float32),
    mesh=_sc_mesh,
    compiler_params=pltpu.CompilerParams(needs_layout_passes=False),
    scratch_types=[
        pltpu.VMEM((_SLAB, _EPB), jnp.int32),
        pltpu.VMEM((_SLAB, _EPB), jnp.int32),
        pltpu.VMEM((_EPB, HID), jnp.float32),
        pltpu.VMEM((_EPB, HID), jnp.float32),
        pltpu.VMEM_SHARED((_NROWS, HID), jnp.float32),
        pltpu.SemaphoreType.DMA,
        pltpu.SemaphoreType.DMA,
    ],
)


def _prep_gcn_body(x_ref, dinv_ref, We_ref, be_ref, W0_ref, h0_ref, g0_ref):
    s0 = jnp.dot(x_ref[...], We_ref[...], preferred_element_type=jnp.float32)
    s0 = s0 + be_ref[...]
    h0 = jnp.dot(s0, W0_ref[...], preferred_element_type=jnp.float32)
    h0_ref[...] = h0
    g = h0 * dinv_ref[...]
    g0_ref[0] = g[:, :HID]
    g0_ref[1] = g[:, HID:]


def _gcn_step_body(acc_ref, h_ref, dinv_ref, b_ref, Wn_ref, hn_ref, gn_ref):
    dinv = dinv_ref[...]
    acc = jnp.concatenate([acc_ref[0], acc_ref[1]], axis=1)
    s = jnp.tanh(dinv * acc + dinv * dinv * h_ref[...] + b_ref[...])
    hn = jnp.dot(s, Wn_ref[...], preferred_element_type=jnp.float32)
    hn_ref[...] = hn
    g = hn * dinv
    gn_ref[0] = g[:, :HID]
    gn_ref[1] = g[:, HID:]


def _gcn_final_body(acc_ref, h_ref, dinv_ref, b_ref, Wo_ref, bo_ref, out_ref):
    dinv = dinv_ref[...]
    acc = jnp.concatenate([acc_ref[0], acc_ref[1]], axis=1)
    s = jnp.tanh(dinv * acc + dinv * dinv * h_ref[...] + b_ref[...])
    out_ref[...] = jnp.dot(s, Wo_ref[...], preferred_element_type=jnp.float32) + bo_ref[...]


def _gat_prep_body(x_ref, Wg_ref, As_ref, Ad_ref, h_ref, exs_ref, asrc_ref, adst_ref):
    h = jnp.dot(x_ref[...], Wg_ref[...], preferred_element_type=jnp.float32)
    h_ref[...] = h
    a_s = jnp.dot(h, As_ref[...], preferred_element_type=jnp.float32)
    a_d = jnp.dot(h, Ad_ref[...], preferred_element_type=jnp.float32)
    asrc_ref[...] = a_s
    adst_ref[...] = a_d
    z = a_s + a_d
    exs_ref[...] = jnp.exp(jnp.where(z > 0, z, 0.2 * z))


def _gat_combine_body(h_ref, exs_ref, num_ref, den_ref, R_ref, M_ref, b_ref,
                      out_ref):
    i = pl.program_id(0)
    real = (i < N_NODES // _ROWS).astype(jnp.float32)
    h = h_ref[...]
    exs3 = exs_ref[...]
    ex48 = jnp.dot(exs3, R_ref[...], preferred_element_type=jnp.float32)
    num = (num_ref[0] + num_ref[1]) * real + h * ex48
    den3 = den_ref[...] * real + exs3
    den48 = jnp.dot(den3, R_ref[...], preferred_element_type=jnp.float32) + 1e-16
    ratio = num / den48
    out_ref[...] = (
        jnp.dot(ratio, M_ref[...], preferred_element_type=jnp.float32) / HEADS
        + b_ref[...]
    )


def _full(shape):
    return pl.BlockSpec(shape, lambda i: (0,) * len(shape))


def _rows(width, blk=_ROWS):
    return pl.BlockSpec((blk, width), lambda i: (i, 0))


def _rows_clamped(width, nblk, blk=_ROWS):
    return pl.BlockSpec((blk, width), lambda i: (jnp.minimum(i, nblk - 1), 0))


@jax.jit
def _run(x, str_init, edge_index, tep_out, W_emb, b_emb, W_gcn0, b_gcn0,
         W_gcn1, b_gcn1, W_out, b_out, W_gat, att_src, att_dst, b_gat):
    src = edge_index[0].astype(jnp.int32)
    dst = edge_index[1].astype(jnp.int32)

    pad = _EPAD - N_EDGES
    srcp = jnp.concatenate([src, jnp.zeros((pad,), jnp.int32)])
    srcp2 = jnp.stack([srcp, srcp + N_NODES]).reshape(_NSC, _EROWS, _EPB)
    dstp = jnp.concatenate(
        [dst, jnp.full((pad,), _TRASH, jnp.int32)]).reshape(_EROWS, _EPB)
    zeros1 = jnp.zeros((_NROWS,), jnp.float32)
    zeros2 = jnp.zeros((_NROWS, HID), jnp.float32)
    ones_h = jnp.ones((_EPB,), jnp.float32)

    deg_parts = _sc_deg(dstp, zeros1, ones_h)
    dinv = jax.lax.rsqrt(
        deg_parts[:N_NODES] + deg_parts[_NROWS:_NROWS + N_NODES] + 1.0)[:, None]

    grid_n = (N_NODES // _ROWS,)
    _g_spec = pl.BlockSpec((_NSC, _ROWS, HID), lambda i: (0, i, 0))
    _acc_spec = pl.BlockSpec((_NSC, _ROWS, HID), lambda i: (0, i, 0))
    _g_shape = jax.ShapeDtypeStruct((_NSC, N_NODES, HID), jnp.float32)

    h0, g0 = pl.pallas_call(
        _prep_gcn_body,
        grid=grid_n,
        in_specs=[_rows(HID), _rows(1), _full((HID, 2 * HID)),
                  _full((2 * HID,)), _full((2 * HID, 2 * HID))],
        out_specs=[_rows(2 * HID), _g_spec],
        out_shape=[jax.ShapeDtypeStruct((N_NODES, 2 * HID), jnp.float32),
                   _g_shape],
    )(str_init, dinv, W_emb, b_emb, W_gcn0)

    acc0 = _sc_scatter(g0.reshape(_NSC * N_NODES, HID), srcp2, dstp, zeros2)

    h1, g1 = pl.pallas_call(
        _gcn_step_body,
        grid=grid_n,
        in_specs=[_acc_spec, _rows(2 * HID), _rows(1),
                  _full((2 * HID,)), _full((2 * HID, 2 * HID))],
        out_specs=[_rows(2 * HID), _g_spec],
        out_shape=[jax.ShapeDtypeStruct((N_NODES, 2 * HID), jnp.float32),
                   _g_shape],
    )(acc0, h0, dinv, b_gcn0, W_gcn1)

    acc1 = _sc_scatter(g1.reshape(_NSC * N_NODES, HID), srcp2, dstp, zeros2)

    str_out = pl.pallas_call(
        _gcn_final_body,
        grid=grid_n,
        in_specs=[_acc_spec, _rows(2 * HID), _rows(1),
                  _full((2 * HID,)), _full((2 * HID, PRED)), _full((PRED,))],
        out_specs=_rows(PRED),
        out_shape=jax.ShapeDtypeStruct((N_NODES, PRED), jnp.float32),
    )(acc1, h1, dinv, b_gcn1, W_out, b_out)

    BN = BATCH * N_NODES
    tep_bn = tep_out.reshape(BN, HID)
    eyeH = jnp.eye(HEADS, dtype=jnp.float32)
    R = jnp.repeat(eyeH, _HP, axis=1)
    att_s48 = jnp.pad(att_src, ((0, 0), (0, _HP - PRED))).reshape(_HF48)
    att_d48 = jnp.pad(att_dst, ((0, 0), (0, _HP - PRED))).reshape(_HF48)
    Asrc = jnp.repeat(eyeH, _HP, axis=0) * att_s48[:, None]
    Adst = jnp.repeat(eyeH, _HP, axis=0) * att_d48[:, None]
    M = jnp.tile(jnp.pad(jnp.eye(PRED, dtype=jnp.float32),
                         ((0, _HP - PRED), (0, 0))), (HEADS, 1))
    W48 = jnp.pad(W_gat.reshape(HID, HEADS, PRED),
                  ((0, 0), (0, 0), (0, _HP - PRED))).reshape(HID, _HF48)

    grid_bn = (BN // _ROWS,)
    h48, exs, a_s, a_d = pl.pallas_call(
        _gat_prep_body,
        grid=grid_bn,
        in_specs=[_rows(HID), _full((HID, _HF48)), _full((_HF48, HEADS)),
                  _full((_HF48, HEADS))],
        out_specs=[_rows(_HF48), _rows(HEADS), _rows(HEADS), _rows(HEADS)],
        out_shape=[jax.ShapeDtypeStruct((BN, _HF48), jnp.float32),
                   jax.ShapeDtypeStruct((BN, HEADS), jnp.float32),
                   jax.ShapeDtypeStruct((BN, HEADS), jnp.float32),
                   jax.ShapeDtypeStruct((BN, HEADS), jnp.float32)],
    )(tep_bn, W48, Asrc, Adst)

    asd = jnp.pad(
        jnp.concatenate([a_s[:N_NODES].T, a_d[:N_NODES].T]),
        ((0, 0), (0, _NROWS - N_NODES))).reshape(-1)
    zeros48 = jnp.zeros((_NROWS, _HF48), jnp.float32)
    num_parts, den_flat = _sc_gat(h48, asd, srcp2[0], dstp, zeros48, zeros1)
    den_p = den_flat.reshape(_NSC, HEADS, _NROWS)
    den3 = (den_p[0] + den_p[1]).T

    spa = pl.pallas_call(
        _gat_combine_body,
        grid=(BN // _ROWS,),
        in_specs=[_rows(_HF48), _rows(HEADS),
                  pl.BlockSpec((_NSC, _ROWS, _HF48),
                               lambda i: (0, jnp.minimum(i, N_NODES // _ROWS - 1), 0)),
                  _rows_clamped(HEADS, N_NODES // _ROWS),
                  _full((HEADS, _HF48)), _full((_HF48, PRED)), _full((PRED,))],
        out_specs=_rows(PRED),
        out_shape=jax.ShapeDtypeStruct((BN, PRED), jnp.float32),
    )(h48, exs, num_parts, den3, R, M, b_gat)

    str_emb = jnp.broadcast_to(str_out[None], (BATCH, N_NODES, PRED))
    return str_emb, spa.reshape(BATCH, N_NODES, PRED)


def kernel(x, str_init, edge_index, tep_out, W_emb, b_emb, W_gcn0, b_gcn0,
           W_gcn1, b_gcn1, W_out, b_out, W_gat, att_src, att_dst, b_gat):
    return _run(x, str_init, edge_index, tep_out, W_emb, b_emb, W_gcn0,
                b_gcn0, W_gcn1, b_gcn1, W_out, b_out, W_gat, att_src,
                att_dst, b_gat)


# speedup vs baseline: 26.5746x; 1.0020x over previous
import functools

import jax
import jax.numpy as jnp
from jax import lax
from jax.experimental import pallas as pl
from jax.experimental.pallas import tpu as pltpu
from jax.experimental.pallas import tpu_sc as plsc

N_NODES = 10000
N_EDGES = 320000
BATCH = 4
HID = 128
PRED = 12
HEADS = 3
HF = HEADS * PRED

_ROWS = 1000

_NSC = 2
_NTILE = 16
_EPB = 128
_EPAD = 327680
_EROWS = _EPAD // _EPB
_TPB = _EROWS // _NTILE
_SLAB = 32
_DPB = _EROWS // (_NSC * _NTILE)
_NROWS = 10240
_RPT = _NROWS // _NTILE
_TRASH = 10100

_sc_mesh = plsc.VectorSubcoreMesh(core_axis_name="c", subcore_axis_name="s")


def _sc_deg_body(dstp, zeros1, ones_h, out, idx_v, ones_v, acc_sh):
    c = lax.axis_index("c")
    s = lax.axis_index("s")
    w = c * _NTILE + s
    pltpu.sync_copy(zeros1.at[pl.ds(s * _RPT, _RPT)],
                    acc_sh.at[pl.ds(s * _RPT, _RPT)])
    pltpu.sync_copy(ones_h, ones_v)
    pltpu.sync_copy(dstp.at[pl.ds(w * _DPB, _DPB)], idx_v)
    plsc.subcore_barrier()

    def body(j, carry):
        pltpu.sync_copy(ones_v, acc_sh.at[idx_v.at[j]], add=True)
        return carry

    lax.fori_loop(0, _DPB, body, 0)
    plsc.subcore_barrier()
    pltpu.sync_copy(acc_sh.at[pl.ds(s * _RPT, _RPT)],
                    out.at[pl.ds(c * _NROWS + s * _RPT, _RPT)])


def _sc_scatter_body(g_hbm, srcp2, dstp, zeros2, out, idx_s, idx_d, rows_a,
                     rows_b, acc_sh, sem_ga, sem_gb, sem_sa, sem_sb):
    c = lax.axis_index("c")
    s = lax.axis_index("s")
    pltpu.sync_copy(zeros2.at[pl.ds(s * _RPT, _RPT)],
                    acc_sh.at[pl.ds(s * _RPT, _RPT)])
    plsc.subcore_barrier()

    npair = _SLAB // 2

    def slab(t, carry):
        start = s * _TPB + t * _SLAB
        pltpu.sync_copy(srcp2.at[c, pl.ds(start, _SLAB)], idx_s)
        pltpu.sync_copy(dstp.at[pl.ds(start, _SLAB)], idx_d)
        pltpu.async_copy(g_hbm.at[idx_s.at[0]], rows_a, sem_ga)

        def pair(j, carry2):
            r0 = 2 * j
            r1 = 2 * j + 1
            pltpu.make_async_copy(g_hbm.at[idx_s.at[r0]], rows_a, sem_ga).wait()

            @pl.when((t + j) > 0)
            def _():
                pltpu.make_async_copy(rows_b, acc_sh.at[idx_d.at[r1]],
                                      sem_sb).wait()

            pltpu.async_copy(g_hbm.at[idx_s.at[r1]], rows_b, sem_gb)
            pltpu.async_copy(rows_a, acc_sh.at[idx_d.at[r0]], sem_sa, add=True)
            pltpu.make_async_copy(g_hbm.at[idx_s.at[r1]], rows_b, sem_gb).wait()
            pltpu.make_async_copy(rows_a, acc_sh.at[idx_d.at[r0]],
                                  sem_sa).wait()

            @pl.when(j < npair - 1)
            def _():
                pltpu.async_copy(g_hbm.at[idx_s.at[r1 + 1]], rows_a, sem_ga)

            pltpu.async_copy(rows_b, acc_sh.at[idx_d.at[r1]], sem_sb, add=True)
            return carry2

        lax.fori_loop(0, npair, pair, 0)
        return carry

    lax.fori_loop(0, _TPB // _SLAB, slab, 0)
    pltpu.make_async_copy(rows_b, acc_sh.at[idx_d.at[_SLAB - 1]],
                          sem_sb).wait()
    plsc.subcore_barrier()
    pltpu.sync_copy(acc_sh.at[pl.ds(s * _RPT, _RPT)],
                    out.at[c, pl.ds(s * _RPT, _RPT)])


_HP = 16
_HF48 = HEADS * _HP


def _sc_gat_body(h48_hbm, asd_hbm, srcp_r, dstp, zeros48, zeros1,
                 num_out, den_out,
                 asd_v, idx_s_v, idx_d_v, rows, exb, num_sh,
                 den0_sh, den1_sh, den2_sh, sem):
    c = lax.axis_index("c")
    s = lax.axis_index("s")
    w = c * _NTILE + s
    dens = (den0_sh, den1_sh, den2_sh)
    pltpu.sync_copy(zeros48.at[pl.ds(s * _RPT, _RPT)],
                    num_sh.at[pl.ds(s * _RPT, _RPT)])
    for h in range(HEADS):
        pltpu.sync_copy(zeros1.at[pl.ds(s * _RPT, _RPT)],
                        dens[h].at[pl.ds(s * _RPT, _RPT)])
    pltpu.sync_copy(srcp_r.at[pl.ds(w * _DPB, _DPB)], idx_s_v)
    pltpu.sync_copy(dstp.at[pl.ds(w * _DPB, _DPB)], idx_d_v)
    pltpu.sync_copy(asd_hbm, asd_v)
    plsc.subcore_barrier()

    def row_body(j, carry):
        pltpu.async_copy(h48_hbm.at[idx_s_v.at[j]], rows, sem).wait()
        for k in range(_EPB // 16):
            si = idx_s_v[j, pl.ds(16 * k, 16)]
            di = idx_d_v[j, pl.ds(16 * k, 16)]
            for h in range(HEADS):
                a1 = plsc.load_gather(asd_v, [si + (h * _NROWS)])
                a2 = plsc.load_gather(asd_v, [di + ((HEADS + h) * _NROWS)])
                z = a1 + a2
                z = jnp.where(z > 0, z, 0.2 * z)
                exb[pl.ds(h * _EPB + 16 * k, 16)] = jnp.exp(z)
        for e in range(_EPB):
            for h in range(HEADS):
                sp = plsc.load_gather(
                    exb, [jnp.full((16,), h * _EPB + e, jnp.int32)])
                rows[e, pl.ds(_HP * h, _HP)] = rows[e, pl.ds(_HP * h, _HP)] * sp
        pltpu.sync_copy(rows, num_sh.at[idx_d_v.at[j]], add=True)
        for h in range(HEADS):
            pltpu.sync_copy(exb.at[pl.ds(h * _EPB, _EPB)],
                            dens[h].at[idx_d_v.at[j]], add=True)
        return carry

    lax.fori_loop(0, _DPB, row_body, 0)
    plsc.subcore_barrier()
    pltpu.sync_copy(num_sh.at[pl.ds(s * _RPT, _RPT)],
                    num_out.at[c, pl.ds(s * _RPT, _RPT)])
    for h in range(HEADS):
        pltpu.sync_copy(
            dens[h].at[pl.ds(s * _RPT, _RPT)],
            den_out.at[pl.ds(c * HEADS * _NROWS + h * _NROWS + s * _RPT,
                             _RPT)])


_sc_gat = pl.kernel(
    _sc_gat_body,
    out_type=(jax.ShapeDtypeStruct((_NSC, _NROWS, _HF48), jnp.float32),
              jax.ShapeDtypeStruct((_NSC * HEADS * _NROWS,), jnp.float32)),
    mesh=_sc_mesh,
    compiler_params=pltpu.CompilerParams(needs_layout_passes=False,
                                         use_tc_tiling_on_sc=False),
    scratch_types=[
        pltpu.VMEM((2 * HEADS * _NROWS,), jnp.float32),
        pltpu.VMEM((_DPB, _EPB), jnp.int32),
        pltpu.VMEM((_DPB, _EPB), jnp.int32),
        pltpu.VMEM((_EPB, _HF48), jnp.float32),
        pltpu.VMEM((HEADS * _EPB,), jnp.float32),
        pltpu.VMEM_SHARED((_NROWS, _HF48), jnp.float32),
        pltpu.VMEM_SHARED((_NROWS,), jnp.float32),
        pltpu.VMEM_SHARED((_NROWS,), jnp.float32),
        pltpu.VMEM_SHARED((_NROWS,), jnp.float32),
        pltpu.SemaphoreType.DMA,
    ],
)


_sc_deg = pl.kernel(
    _sc_deg_body,
    out_type=jax.ShapeDtypeStruct((_NSC * _NROWS,), jnp.float32),
    mesh=_sc_mesh,
    compiler_params=pltpu.CompilerParams(needs_layout_passes=False),
    scratch_types=[
        pltpu.VMEM((_DPB, _EPB), jnp.int32),
        pltpu.VMEM((_EPB,), jnp.float32),
        pltpu.VMEM_SHARED((_NROWS,), jnp.float32),
    ],
)

_sc_scatter = pl.kernel(
    _sc_scatter_body,
    out_type=jax.ShapeDtypeStruct((_NSC, _NROWS, HID), jnp.float32),
    mesh=_sc_mesh,
    compiler_params=pltpu.CompilerParams(needs_layout_passes=False),
    scratch_types=[
        pltpu.VMEM((_SLAB, _EPB), jnp.int32),
        pltpu.VMEM((_SLAB, _EPB), jnp.int32),
        pltpu.VMEM((_EPB, HID), jnp.float32),
        pltpu.VMEM((_EPB, HID), jnp.float32),
        pltpu.VMEM_SHARED((_NROWS, HID), jnp.float32),
        pltpu.SemaphoreType.DMA,
        pltpu.SemaphoreType.DMA,
        pltpu.SemaphoreType.DMA,
        pltpu.SemaphoreType.DMA,
    ],
)


def _prep_gcn_body(x_ref, dinv_ref, We_ref, be_ref, W0_ref, h0_ref, g0_ref):
    s0 = jnp.dot(x_ref[...], We_ref[...], preferred_element_type=jnp.float32)
    s0 = s0 + be_ref[...]
    h0 = jnp.dot(s0, W0_ref[...], preferred_element_type=jnp.float32)
    h0_ref[...] = h0
    g = h0 * dinv_ref[...]
    g0_ref[0] = g[:, :HID]
    g0_ref[1] = g[:, HID:]


def _gcn_step_body(acc_ref, h_ref, dinv_ref, b_ref, Wn_ref, hn_ref, gn_ref):
    dinv = dinv_ref[...]
    acc = jnp.concatenate([acc_ref[0], acc_ref[1]], axis=1)
    s = jnp.tanh(dinv * acc + dinv * dinv * h_ref[...] + b_ref[...])
    hn = jnp.dot(s, Wn_ref[...], preferred_element_type=jnp.float32)
    hn_ref[...] = hn
    g = hn * dinv
    gn_ref[0] = g[:, :HID]
    gn_ref[1] = g[:, HID:]


def _gcn_final_body(acc_ref, h_ref, dinv_ref, b_ref, Wo_ref, bo_ref, out_ref):
    dinv = dinv_ref[...]
    acc = jnp.concatenate([acc_ref[0], acc_ref[1]], axis=1)
    s = jnp.tanh(dinv * acc + dinv * dinv * h_ref[...] + b_ref[...])
    out_ref[...] = jnp.dot(s, Wo_ref[...], preferred_element_type=jnp.float32) + bo_ref[...]


def _gat_prep_body(x_ref, Wg_ref, As_ref, Ad_ref, h_ref, exs_ref, asrc_ref, adst_ref):
    h = jnp.dot(x_ref[...], Wg_ref[...], preferred_element_type=jnp.float32)
    h_ref[...] = h
    a_s = jnp.dot(h, As_ref[...], preferred_element_type=jnp.float32)
    a_d = jnp.dot(h, Ad_ref[...], preferred_element_type=jnp.float32)
    asrc_ref[...] = a_s
    adst_ref[...] = a_d
    z = a_s + a_d
    exs_ref[...] = jnp.exp(jnp.where(z > 0, z, 0.2 * z))


def _gat_combine_body(h_ref, exs_ref, num_ref, den_ref, R_ref, M_ref, b_ref,
                      out_ref):
    i = pl.program_id(0)
    real = (i < N_NODES // _ROWS).astype(jnp.float32)
    h = h_ref[...]
    exs3 = exs_ref[...]
    ex48 = jnp.dot(exs3, R_ref[...], preferred_element_type=jnp.float32)
    num = (num_ref[0] + num_ref[1]) * real + h * ex48
    den3 = den_ref[...] * real + exs3
    den48 = jnp.dot(den3, R_ref[...], preferred_element_type=jnp.float32) + 1e-16
    ratio = num / den48
    out_ref[...] = (
        jnp.dot(ratio, M_ref[...], preferred_element_type=jnp.float32) / HEADS
        + b_ref[...]
    )


def _full(shape):
    return pl.BlockSpec(shape, lambda i: (0,) * len(shape))


def _rows(width, blk=_ROWS):
    return pl.BlockSpec((blk, width), lambda i: (i, 0))


def _rows_clamped(width, nblk, blk=_ROWS):
    return pl.BlockSpec((blk, width), lambda i: (jnp.minimum(i, nblk - 1), 0))


@jax.jit
def _run(x, str_init, edge_index, tep_out, W_emb, b_emb, W_gcn0, b_gcn0,
         W_gcn1, b_gcn1, W_out, b_out, W_gat, att_src, att_dst, b_gat):
    src = edge_index[0].astype(jnp.int32)
    dst = edge_index[1].astype(jnp.int32)

    pad = _EPAD - N_EDGES
    srcp = jnp.concatenate([src, jnp.zeros((pad,), jnp.int32)])
    srcp2 = jnp.stack([srcp, srcp + N_NODES]).reshape(_NSC, _EROWS, _EPB)
    dstp = jnp.concatenate(
        [dst, jnp.full((pad,), _TRASH, jnp.int32)]).reshape(_EROWS, _EPB)
    zeros1 = jnp.zeros((_NROWS,), jnp.float32)
    zeros2 = jnp.zeros((_NROWS, HID), jnp.float32)
    ones_h = jnp.ones((_EPB,), jnp.float32)

    deg_parts = _sc_deg(dstp, zeros1, ones_h)
    dinv = jax.lax.rsqrt(
        deg_parts[:N_NODES] + deg_parts[_NROWS:_NROWS + N_NODES] + 1.0)[:, None]

    grid_n = (N_NODES // _ROWS,)
    _g_spec = pl.BlockSpec((_NSC, _ROWS, HID), lambda i: (0, i, 0))
    _acc_spec = pl.BlockSpec((_NSC, _ROWS, HID), lambda i: (0, i, 0))
    _g_shape = jax.ShapeDtypeStruct((_NSC, N_NODES, HID), jnp.float32)

    h0, g0 = pl.pallas_call(
        _prep_gcn_body,
        grid=grid_n,
        in_specs=[_rows(HID), _rows(1), _full((HID, 2 * HID)),
                  _full((2 * HID,)), _full((2 * HID, 2 * HID))],
        out_specs=[_rows(2 * HID), _g_spec],
        out_shape=[jax.ShapeDtypeStruct((N_NODES, 2 * HID), jnp.float32),
                   _g_shape],
    )(str_init, dinv, W_emb, b_emb, W_gcn0)

    acc0 = _sc_scatter(g0.reshape(_NSC * N_NODES, HID), srcp2, dstp, zeros2)

    h1, g1 = pl.pallas_call(
        _gcn_step_body,
        grid=grid_n,
        in_specs=[_acc_spec, _rows(2 * HID), _rows(1),
                  _full((2 * HID,)), _full((2 * HID, 2 * HID))],
        out_specs=[_rows(2 * HID), _g_spec],
        out_shape=[jax.ShapeDtypeStruct((N_NODES, 2 * HID), jnp.float32),
                   _g_shape],
    )(acc0, h0, dinv, b_gcn0, W_gcn1)

    acc1 = _sc_scatter(g1.reshape(_NSC * N_NODES, HID), srcp2, dstp, zeros2)

    str_out = pl.pallas_call(
        _gcn_final_body,
        grid=grid_n,
        in_specs=[_acc_spec, _rows(2 * HID), _rows(1),
                  _full((2 * HID,)), _full((2 * HID, PRED)), _full((PRED,))],
        out_specs=_rows(PRED),
        out_shape=jax.ShapeDtypeStruct((N_NODES, PRED), jnp.float32),
    )(acc1, h1, dinv, b_gcn1, W_out, b_out)

    BN = BATCH * N_NODES
    tep_bn = tep_out.reshape(BN, HID)
    eyeH = jnp.eye(HEADS, dtype=jnp.float32)
    R = jnp.repeat(eyeH, _HP, axis=1)
    att_s48 = jnp.pad(att_src, ((0, 0), (0, _HP - PRED))).reshape(_HF48)
    att_d48 = jnp.pad(att_dst, ((0, 0), (0, _HP - PRED))).reshape(_HF48)
    Asrc = jnp.repeat(eyeH, _HP, axis=0) * att_s48[:, None]
    Adst = jnp.repeat(eyeH, _HP, axis=0) * att_d48[:, None]
    M = jnp.tile(jnp.pad(jnp.eye(PRED, dtype=jnp.float32),
                         ((0, _HP - PRED), (0, 0))), (HEADS, 1))
    W48 = jnp.pad(W_gat.reshape(HID, HEADS, PRED),
                  ((0, 0), (0, 0), (0, _HP - PRED))).reshape(HID, _HF48)

    grid_bn = (BN // _ROWS,)
    h48, exs, a_s, a_d = pl.pallas_call(
        _gat_prep_body,
        grid=grid_bn,
        in_specs=[_rows(HID), _full((HID, _HF48)), _full((_HF48, HEADS)),
                  _full((_HF48, HEADS))],
        out_specs=[_rows(_HF48), _rows(HEADS), _rows(HEADS), _rows(HEADS)],
        out_shape=[jax.ShapeDtypeStruct((BN, _HF48), jnp.float32),
                   jax.ShapeDtypeStruct((BN, HEADS), jnp.float32),
                   jax.ShapeDtypeStruct((BN, HEADS), jnp.float32),
                   jax.ShapeDtypeStruct((BN, HEADS), jnp.float32)],
    )(tep_bn, W48, Asrc, Adst)

    asd = jnp.pad(
        jnp.concatenate([a_s[:N_NODES].T, a_d[:N_NODES].T]),
        ((0, 0), (0, _NROWS - N_NODES))).reshape(-1)
    zeros48 = jnp.zeros((_NROWS, _HF48), jnp.float32)
    num_parts, den_flat = _sc_gat(h48, asd, srcp2[0], dstp, zeros48, zeros1)
    den_p = den_flat.reshape(_NSC, HEADS, _NROWS)
    den3 = (den_p[0] + den_p[1]).T

    spa = pl.pallas_call(
        _gat_combine_body,
        grid=(BN // _ROWS,),
        in_specs=[_rows(_HF48), _rows(HEADS),
                  pl.BlockSpec((_NSC, _ROWS, _HF48),
                               lambda i: (0, jnp.minimum(i, N_NODES // _ROWS - 1), 0)),
                  _rows_clamped(HEADS, N_NODES // _ROWS),
                  _full((HEADS, _HF48)), _full((_HF48, PRED)), _full((PRED,))],
        out_specs=_rows(PRED),
        out_shape=jax.ShapeDtypeStruct((BN, PRED), jnp.float32),
    )(h48, exs, num_parts, den3, R, M, b_gat)

    str_emb = jnp.broadcast_to(str_out[None], (BATCH, N_NODES, PRED))
    return str_emb, spa.reshape(BATCH, N_NODES, PRED)


def kernel(x, str_init, edge_index, tep_out, W_emb, b_emb, W_gcn0, b_gcn0,
           W_gcn1, b_gcn1, W_out, b_out, W_gat, att_src, att_dst, b_gat):
    return _run(x, str_init, edge_index, tep_out, W_emb, b_emb, W_gcn0,
                b_gcn0, W_gcn1, b_gcn1, W_out, b_out, W_gat, att_src,
                att_dst, b_gat)


# speedup vs baseline: 28.6591x; 1.0784x over previous
import functools

import jax
import jax.numpy as jnp
from jax import lax
from jax.experimental import pallas as pl
from jax.experimental.pallas import tpu as pltpu
from jax.experimental.pallas import tpu_sc as plsc

N_NODES = 10000
N_EDGES = 320000
BATCH = 4
HID = 128
PRED = 12
HEADS = 3
HF = HEADS * PRED

_ROWS = 1000

_NSC = 2
_NTILE = 16
_EPB = 128
_EPAD = 327680
_EROWS = _EPAD // _EPB
_TPB = _EROWS // _NTILE
_SLAB = 32
_DPB = _EROWS // (_NSC * _NTILE)
_NROWS = 10240
_RPT = _NROWS // _NTILE
_TRASH = 10100

_sc_mesh = plsc.VectorSubcoreMesh(core_axis_name="c", subcore_axis_name="s")


def _sc_deg_body(dstp, zeros1, ones_h, out, idx_v, ones_v, acc_sh):
    c = lax.axis_index("c")
    s = lax.axis_index("s")
    w = c * _NTILE + s
    pltpu.sync_copy(zeros1.at[pl.ds(s * _RPT, _RPT)],
                    acc_sh.at[pl.ds(s * _RPT, _RPT)])
    pltpu.sync_copy(ones_h, ones_v)
    pltpu.sync_copy(dstp.at[pl.ds(w * _DPB, _DPB)], idx_v)
    plsc.subcore_barrier()

    def body(j, carry):
        pltpu.sync_copy(ones_v, acc_sh.at[idx_v.at[j]], add=True)
        return carry

    lax.fori_loop(0, _DPB, body, 0)
    plsc.subcore_barrier()
    pltpu.sync_copy(acc_sh.at[pl.ds(s * _RPT, _RPT)],
                    out.at[pl.ds(c * _NROWS + s * _RPT, _RPT)])


def _sc_scatter_body(g_hbm, srcp2, dstp, zeros2, out, idx_s, idx_d, rows_a,
                     rows_b, acc_sh, sem_ga, sem_gb, sem_sa, sem_sb):
    c = lax.axis_index("c")
    s = lax.axis_index("s")
    pltpu.sync_copy(zeros2.at[pl.ds(s * _RPT, _RPT)],
                    acc_sh.at[pl.ds(s * _RPT, _RPT)])
    plsc.subcore_barrier()

    npair = _SLAB // 2

    def slab(t, carry):
        start = s * _TPB + t * _SLAB
        pltpu.sync_copy(srcp2.at[c, pl.ds(start, _SLAB)], idx_s)
        pltpu.sync_copy(dstp.at[pl.ds(start, _SLAB)], idx_d)
        pltpu.async_copy(g_hbm.at[idx_s.at[0]], rows_a, sem_ga)

        def pair(j, carry2):
            r0 = 2 * j
            r1 = 2 * j + 1
            pltpu.make_async_copy(g_hbm.at[idx_s.at[r0]], rows_a, sem_ga).wait()

            @pl.when((t + j) > 0)
            def _():
                pltpu.make_async_copy(rows_b, acc_sh.at[idx_d.at[r1]],
                                      sem_sb).wait()

            pltpu.async_copy(g_hbm.at[idx_s.at[r1]], rows_b, sem_gb)
            pltpu.async_copy(rows_a, acc_sh.at[idx_d.at[r0]], sem_sa, add=True)
            pltpu.make_async_copy(g_hbm.at[idx_s.at[r1]], rows_b, sem_gb).wait()
            pltpu.make_async_copy(rows_a, acc_sh.at[idx_d.at[r0]],
                                  sem_sa).wait()

            @pl.when(j < npair - 1)
            def _():
                pltpu.async_copy(g_hbm.at[idx_s.at[r1 + 1]], rows_a, sem_ga)

            pltpu.async_copy(rows_b, acc_sh.at[idx_d.at[r1]], sem_sb, add=True)
            return carry2

        lax.fori_loop(0, npair, pair, 0)
        return carry

    lax.fori_loop(0, _TPB // _SLAB, slab, 0)
    pltpu.make_async_copy(rows_b, acc_sh.at[idx_d.at[_SLAB - 1]],
                          sem_sb).wait()
    plsc.subcore_barrier()
    pltpu.sync_copy(acc_sh.at[pl.ds(s * _RPT, _RPT)],
                    out.at[c, pl.ds(s * _RPT, _RPT)])


_HP = 16
_HF48 = HEADS * _HP


def _sc_gat_body(h48_hbm, asd_hbm, srcp_r, dstp, zeros48, zeros1,
                 num_out, den_out,
                 asd_v, idx_s_v, idx_d_v, rows_a, rows_b, exb_a, exb_b,
                 num_sh, den0_sh, den1_sh, den2_sh,
                 sem_ga, sem_gb, sem_sa, sem_sb, sem_da, sem_db):
    c = lax.axis_index("c")
    s = lax.axis_index("s")
    w = c * _NTILE + s
    dens = (den0_sh, den1_sh, den2_sh)
    pltpu.sync_copy(zeros48.at[pl.ds(s * _RPT, _RPT)],
                    num_sh.at[pl.ds(s * _RPT, _RPT)])
    for h in range(HEADS):
        pltpu.sync_copy(zeros1.at[pl.ds(s * _RPT, _RPT)],
                        dens[h].at[pl.ds(s * _RPT, _RPT)])
    pltpu.sync_copy(srcp_r.at[pl.ds(w * _DPB, _DPB)], idx_s_v)
    pltpu.sync_copy(dstp.at[pl.ds(w * _DPB, _DPB)], idx_d_v)
    pltpu.sync_copy(asd_hbm, asd_v)
    plsc.subcore_barrier()

    def compute(j, rows, exb):
        for k in range(_EPB // 16):
            si = idx_s_v[j, pl.ds(16 * k, 16)]
            di = idx_d_v[j, pl.ds(16 * k, 16)]
            for h in range(HEADS):
                a1 = plsc.load_gather(asd_v, [si + (h * _NROWS)])
                a2 = plsc.load_gather(asd_v, [di + ((HEADS + h) * _NROWS)])
                z = a1 + a2
                z = jnp.where(z > 0, z, 0.2 * z)
                exb[pl.ds(h * _EPB + 16 * k, 16)] = jnp.exp(z)
        for e in range(_EPB):
            for h in range(HEADS):
                sp = plsc.load_gather(
                    exb, [jnp.full((16,), h * _EPB + e, jnp.int32)])
                rows[e, pl.ds(_HP * h, _HP)] = rows[e, pl.ds(_HP * h, _HP)] * sp

    def issue_scatters(j, rows, exb, sem_s, sem_d):
        pltpu.async_copy(rows, num_sh.at[idx_d_v.at[j]], sem_s, add=True)
        for h in range(HEADS):
            pltpu.async_copy(exb.at[pl.ds(h * _EPB, _EPB)],
                             dens[h].at[idx_d_v.at[j]], sem_d, add=True)

    def wait_scatters(j, rows, exb, sem_s, sem_d):
        pltpu.make_async_copy(rows, num_sh.at[idx_d_v.at[j]], sem_s).wait()
        for h in range(HEADS):
            pltpu.make_async_copy(exb.at[pl.ds(h * _EPB, _EPB)],
                                  dens[h].at[idx_d_v.at[j]], sem_d).wait()

    npair = _DPB // 2
    pltpu.async_copy(h48_hbm.at[idx_s_v.at[0]], rows_a, sem_ga)

    def pair(j, carry):
        r0 = 2 * j
        r1 = 2 * j + 1
        pltpu.make_async_copy(h48_hbm.at[idx_s_v.at[r0]], rows_a,
                              sem_ga).wait()

        @pl.when(j > 0)
        def _():
            wait_scatters(r1, rows_b, exb_b, sem_sb, sem_db)

        pltpu.async_copy(h48_hbm.at[idx_s_v.at[r1]], rows_b, sem_gb)
        compute(r0, rows_a, exb_a)
        issue_scatters(r0, rows_a, exb_a, sem_sa, sem_da)
        pltpu.make_async_copy(h48_hbm.at[idx_s_v.at[r1]], rows_b,
                              sem_gb).wait()
        wait_scatters(r0, rows_a, exb_a, sem_sa, sem_da)

        @pl.when(j < npair - 1)
        def _():
            pltpu.async_copy(h48_hbm.at[idx_s_v.at[r1 + 1]], rows_a, sem_ga)

        compute(r1, rows_b, exb_b)
        issue_scatters(r1, rows_b, exb_b, sem_sb, sem_db)
        return carry

    lax.fori_loop(0, npair, pair, 0)
    wait_scatters(_DPB - 1, rows_b, exb_b, sem_sb, sem_db)
    plsc.subcore_barrier()
    pltpu.sync_copy(num_sh.at[pl.ds(s * _RPT, _RPT)],
                    num_out.at[c, pl.ds(s * _RPT, _RPT)])
    for h in range(HEADS):
        pltpu.sync_copy(
            dens[h].at[pl.ds(s * _RPT, _RPT)],
            den_out.at[pl.ds(c * HEADS * _NROWS + h * _NROWS + s * _RPT,
                             _RPT)])


_sc_gat = pl.kernel(
    _sc_gat_body,
    out_type=(jax.ShapeDtypeStruct((_NSC, _NROWS, _HF48), jnp.float32),
              jax.ShapeDtypeStruct((_NSC * HEADS * _NROWS,), jnp.float32)),
    mesh=_sc_mesh,
    compiler_params=pltpu.CompilerParams(needs_layout_passes=False,
                                         use_tc_tiling_on_sc=False),
    scratch_types=[
        pltpu.VMEM((2 * HEADS * _NROWS,), jnp.float32),
        pltpu.VMEM((_DPB, _EPB), jnp.int32),
        pltpu.VMEM((_DPB, _EPB), jnp.int32),
        pltpu.VMEM((_EPB, _HF48), jnp.float32),
        pltpu.VMEM((_EPB, _HF48), jnp.float32),
        pltpu.VMEM((HEADS * _EPB,), jnp.float32),
        pltpu.VMEM((HEADS * _EPB,), jnp.float32),
        pltpu.VMEM_SHARED((_NROWS, _HF48), jnp.float32),
        pltpu.VMEM_SHARED((_NROWS,), jnp.float32),
        pltpu.VMEM_SHARED((_NROWS,), jnp.float32),
        pltpu.VMEM_SHARED((_NROWS,), jnp.float32),
        pltpu.SemaphoreType.DMA,
        pltpu.SemaphoreType.DMA,
        pltpu.SemaphoreType.DMA,
        pltpu.SemaphoreType.DMA,
        pltpu.SemaphoreType.DMA,
        pltpu.SemaphoreType.DMA,
    ],
)


_sc_deg = pl.kernel(
    _sc_deg_body,
    out_type=jax.ShapeDtypeStruct((_NSC * _NROWS,), jnp.float32),
    mesh=_sc_mesh,
    compiler_params=pltpu.CompilerParams(needs_layout_passes=False),
    scratch_types=[
        pltpu.VMEM((_DPB, _EPB), jnp.int32),
        pltpu.VMEM((_EPB,), jnp.float32),
        pltpu.VMEM_SHARED((_NROWS,), jnp.float32),
    ],
)

_sc_scatter = pl.kernel(
    _sc_scatter_body,
    out_type=jax.ShapeDtypeStruct((_NSC, _NROWS, HID), jnp.float32),
    mesh=_sc_mesh,
    compiler_params=pltpu.CompilerParams(needs_layout_passes=False),
    scratch_types=[
        pltpu.VMEM((_SLAB, _EPB), jnp.int32),
        pltpu.VMEM((_SLAB, _EPB), jnp.int32),
        pltpu.VMEM((_EPB, HID), jnp.float32),
        pltpu.VMEM((_EPB, HID), jnp.float32),
        pltpu.VMEM_SHARED((_NROWS, HID), jnp.float32),
        pltpu.SemaphoreType.DMA,
        pltpu.SemaphoreType.DMA,
        pltpu.SemaphoreType.DMA,
        pltpu.SemaphoreType.DMA,
    ],
)


def _prep_gcn_body(x_ref, dinv_ref, We_ref, be_ref, W0_ref, h0_ref, g0_ref):
    s0 = jnp.dot(x_ref[...], We_ref[...], preferred_element_type=jnp.float32)
    s0 = s0 + be_ref[...]
    h0 = jnp.dot(s0, W0_ref[...], preferred_element_type=jnp.float32)
    h0_ref[...] = h0
    g = h0 * dinv_ref[...]
    g0_ref[0] = g[:, :HID]
    g0_ref[1] = g[:, HID:]


def _gcn_step_body(acc_ref, h_ref, dinv_ref, b_ref, Wn_ref, hn_ref, gn_ref):
    dinv = dinv_ref[...]
    acc = jnp.concatenate([acc_ref[0], acc_ref[1]], axis=1)
    s = jnp.tanh(dinv * acc + dinv * dinv * h_ref[...] + b_ref[...])
    hn = jnp.dot(s, Wn_ref[...], preferred_element_type=jnp.float32)
    hn_ref[...] = hn
    g = hn * dinv
    gn_ref[0] = g[:, :HID]
    gn_ref[1] = g[:, HID:]


def _gcn_final_body(acc_ref, h_ref, dinv_ref, b_ref, Wo_ref, bo_ref, out_ref):
    dinv = dinv_ref[...]
    acc = jnp.concatenate([acc_ref[0], acc_ref[1]], axis=1)
    s = jnp.tanh(dinv * acc + dinv * dinv * h_ref[...] + b_ref[...])
    out_ref[...] = jnp.dot(s, Wo_ref[...], preferred_element_type=jnp.float32) + bo_ref[...]


def _gat_prep_body(x_ref, Wg_ref, As_ref, Ad_ref, h_ref, exs_ref, asrc_ref, adst_ref):
    h = jnp.dot(x_ref[...], Wg_ref[...], preferred_element_type=jnp.float32)
    h_ref[...] = h
    a_s = jnp.dot(h, As_ref[...], preferred_element_type=jnp.float32)
    a_d = jnp.dot(h, Ad_ref[...], preferred_element_type=jnp.float32)
    asrc_ref[...] = a_s
    adst_ref[...] = a_d
    z = a_s + a_d
    exs_ref[...] = jnp.exp(jnp.where(z > 0, z, 0.2 * z))


def _gat_combine_body(h_ref, exs_ref, num_ref, den_ref, R_ref, M_ref, b_ref,
                      out_ref):
    i = pl.program_id(0)
    real = (i < N_NODES // _ROWS).astype(jnp.float32)
    h = h_ref[...]
    exs3 = exs_ref[...]
    ex48 = jnp.dot(exs3, R_ref[...], preferred_element_type=jnp.float32)
    num = (num_ref[0] + num_ref[1]) * real + h * ex48
    den3 = den_ref[...] * real + exs3
    den48 = jnp.dot(den3, R_ref[...], preferred_element_type=jnp.float32) + 1e-16
    ratio = num / den48
    out_ref[...] = (
        jnp.dot(ratio, M_ref[...], preferred_element_type=jnp.float32) / HEADS
        + b_ref[...]
    )


def _full(shape):
    return pl.BlockSpec(shape, lambda i: (0,) * len(shape))


def _rows(width, blk=_ROWS):
    return pl.BlockSpec((blk, width), lambda i: (i, 0))


def _rows_clamped(width, nblk, blk=_ROWS):
    return pl.BlockSpec((blk, width), lambda i: (jnp.minimum(i, nblk - 1), 0))


@jax.jit
def _run(x, str_init, edge_index, tep_out, W_emb, b_emb, W_gcn0, b_gcn0,
         W_gcn1, b_gcn1, W_out, b_out, W_gat, att_src, att_dst, b_gat):
    src = edge_index[0].astype(jnp.int32)
    dst = edge_index[1].astype(jnp.int32)

    pad = _EPAD - N_EDGES
    srcp = jnp.concatenate([src, jnp.zeros((pad,), jnp.int32)])
    srcp2 = jnp.stack([srcp, srcp + N_NODES]).reshape(_NSC, _EROWS, _EPB)
    dstp = jnp.concatenate(
        [dst, jnp.full((pad,), _TRASH, jnp.int32)]).reshape(_EROWS, _EPB)
    zeros1 = jnp.zeros((_NROWS,), jnp.float32)
    zeros2 = jnp.zeros((_NROWS, HID), jnp.float32)
    ones_h = jnp.ones((_EPB,), jnp.float32)

    deg_parts = _sc_deg(dstp, zeros1, ones_h)
    dinv = jax.lax.rsqrt(
        deg_parts[:N_NODES] + deg_parts[_NROWS:_NROWS + N_NODES] + 1.0)[:, None]

    grid_n = (N_NODES // _ROWS,)
    _g_spec = pl.BlockSpec((_NSC, _ROWS, HID), lambda i: (0, i, 0))
    _acc_spec = pl.BlockSpec((_NSC, _ROWS, HID), lambda i: (0, i, 0))
    _g_shape = jax.ShapeDtypeStruct((_NSC, N_NODES, HID), jnp.float32)

    h0, g0 = pl.pallas_call(
        _prep_gcn_body,
        grid=grid_n,
        in_specs=[_rows(HID), _rows(1), _full((HID, 2 * HID)),
                  _full((2 * HID,)), _full((2 * HID, 2 * HID))],
        out_specs=[_rows(2 * HID), _g_spec],
        out_shape=[jax.ShapeDtypeStruct((N_NODES, 2 * HID), jnp.float32),
                   _g_shape],
    )(str_init, dinv, W_emb, b_emb, W_gcn0)

    acc0 = _sc_scatter(g0.reshape(_NSC * N_NODES, HID), srcp2, dstp, zeros2)

    h1, g1 = pl.pallas_call(
        _gcn_step_body,
        grid=grid_n,
        in_specs=[_acc_spec, _rows(2 * HID), _rows(1),
                  _full((2 * HID,)), _full((2 * HID, 2 * HID))],
        out_specs=[_rows(2 * HID), _g_spec],
        out_shape=[jax.ShapeDtypeStruct((N_NODES, 2 * HID), jnp.float32),
                   _g_shape],
    )(acc0, h0, dinv, b_gcn0, W_gcn1)

    acc1 = _sc_scatter(g1.reshape(_NSC * N_NODES, HID), srcp2, dstp, zeros2)

    str_out = pl.pallas_call(
        _gcn_final_body,
        grid=grid_n,
        in_specs=[_acc_spec, _rows(2 * HID), _rows(1),
                  _full((2 * HID,)), _full((2 * HID, PRED)), _full((PRED,))],
        out_specs=_rows(PRED),
        out_shape=jax.ShapeDtypeStruct((N_NODES, PRED), jnp.float32),
    )(acc1, h1, dinv, b_gcn1, W_out, b_out)

    BN = BATCH * N_NODES
    tep_bn = tep_out.reshape(BN, HID)
    eyeH = jnp.eye(HEADS, dtype=jnp.float32)
    R = jnp.repeat(eyeH, _HP, axis=1)
    att_s48 = jnp.pad(att_src, ((0, 0), (0, _HP - PRED))).reshape(_HF48)
    att_d48 = jnp.pad(att_dst, ((0, 0), (0, _HP - PRED))).reshape(_HF48)
    Asrc = jnp.repeat(eyeH, _HP, axis=0) * att_s48[:, None]
    Adst = jnp.repeat(eyeH, _HP, axis=0) * att_d48[:, None]
    M = jnp.tile(jnp.pad(jnp.eye(PRED, dtype=jnp.float32),
                         ((0, _HP - PRED), (0, 0))), (HEADS, 1))
    W48 = jnp.pad(W_gat.reshape(HID, HEADS, PRED),
                  ((0, 0), (0, 0), (0, _HP - PRED))).reshape(HID, _HF48)

    grid_bn = (BN // _ROWS,)
    h48, exs, a_s, a_d = pl.pallas_call(
        _gat_prep_body,
        grid=grid_bn,
        in_specs=[_rows(HID), _full((HID, _HF48)), _full((_HF48, HEADS)),
                  _full((_HF48, HEADS))],
        out_specs=[_rows(_HF48), _rows(HEADS), _rows(HEADS), _rows(HEADS)],
        out_shape=[jax.ShapeDtypeStruct((BN, _HF48), jnp.float32),
                   jax.ShapeDtypeStruct((BN, HEADS), jnp.float32),
                   jax.ShapeDtypeStruct((BN, HEADS), jnp.float32),
                   jax.ShapeDtypeStruct((BN, HEADS), jnp.float32)],
    )(tep_bn, W48, Asrc, Adst)

    asd = jnp.pad(
        jnp.concatenate([a_s[:N_NODES].T, a_d[:N_NODES].T]),
        ((0, 0), (0, _NROWS - N_NODES))).reshape(-1)
    zeros48 = jnp.zeros((_NROWS, _HF48), jnp.float32)
    num_parts, den_flat = _sc_gat(h48, asd, srcp2[0], dstp, zeros48, zeros1)
    den_p = den_flat.reshape(_NSC, HEADS, _NROWS)
    den3 = (den_p[0] + den_p[1]).T

    spa = pl.pallas_call(
        _gat_combine_body,
        grid=(BN // _ROWS,),
        in_specs=[_rows(_HF48), _rows(HEADS),
                  pl.BlockSpec((_NSC, _ROWS, _HF48),
                               lambda i: (0, jnp.minimum(i, N_NODES // _ROWS - 1), 0)),
                  _rows_clamped(HEADS, N_NODES // _ROWS),
                  _full((HEADS, _HF48)), _full((_HF48, PRED)), _full((PRED,))],
        out_specs=_rows(PRED),
        out_shape=jax.ShapeDtypeStruct((BN, PRED), jnp.float32),
    )(h48, exs, num_parts, den3, R, M, b_gat)

    str_emb = jnp.broadcast_to(str_out[None], (BATCH, N_NODES, PRED))
    return str_emb, spa.reshape(BATCH, N_NODES, PRED)


def kernel(x, str_init, edge_index, tep_out, W_emb, b_emb, W_gcn0, b_gcn0,
           W_gcn1, b_gcn1, W_out, b_out, W_gat, att_src, att_dst, b_gat):
    return _run(x, str_init, edge_index, tep_out, W_emb, b_emb, W_gcn0,
                b_gcn0, W_gcn1, b_gcn1, W_out, b_out, W_gat, att_src,
                att_dst, b_gat)
